# Initial kernel scaffold; baseline (speedup 1.0000x reference)
import jax
import jax.numpy as jnp
from jax.experimental import pallas as pl


def kernel(core_w, core_b, core_wg, core_bg, ep_w1_post, ep_w1_cur, ep_w1_glob, ep_b1, el_w1_cur, el_w1_post, el_w1_glob, el_b1, el_w2, el_b2, el_w3, el_b3, wout_el, wout_ep, bout, node_counts, node_offsets, graph_ids, node_features, partner_index_index, partner_index_values):
    raise NotImplementedError("write your pallas kernel here")


# speedup vs baseline: 1.3467x; 1.3467x over previous
import functools

import jax
import jax.numpy as jnp
from jax import lax
from jax.experimental import pallas as pl
from jax.experimental.pallas import tpu as pltpu

EMBED_DIM = 32
NODE_FEAT_DIM = 16
NUM_EDGE_TYPES = 8
NUM_GRAPHS = 1024
ROW_TILE = 2048
P_TILE = 2048


def _compiler_params():
    return pltpu.CompilerParams(
        dimension_semantics=("parallel",),
        vmem_limit_bytes=64 * 1024 * 1024,
    )


def _pinned(shape):
    return pl.BlockSpec(shape, lambda i: tuple(0 for _ in shape))


def _partner_kernel(gid_ref, nf_ref, wcat_ref, corew_ref, coreb_ref,
                    w2_ref, b2_ref, out_ref, *, num_graphs):
    tile = nf_ref.shape[0]
    x = jnp.maximum(
        jnp.dot(nf_ref[...].astype(jnp.bfloat16), corew_ref[...],
                preferred_element_type=jnp.float32) + coreb_ref[...], 0.0)
    gid = gid_ref[...]
    giota = lax.broadcasted_iota(jnp.int32, (tile, num_graphs), 1)
    onehot = (gid == giota).astype(jnp.bfloat16)
    lhs = jnp.concatenate([onehot, x.astype(jnp.bfloat16)], axis=1)
    h = jnp.maximum(
        jnp.dot(lhs, wcat_ref[...], preferred_element_type=jnp.float32), 0.0)
    out_ref[...] = (jnp.dot(h.astype(jnp.bfloat16), w2_ref[...],
                            preferred_element_type=jnp.float32) + b2_ref[...])


def _label_kernel(pidx_ref, pnf_ref, precat_ref, corew_ref, coreb_ref,
                  w2_ref, b2_ref, w3_ref, b3_ref, out_ref, *, num_graphs):
    tile = pnf_ref.shape[0]
    par = jnp.maximum(
        jnp.dot(pnf_ref[...].astype(jnp.bfloat16), corew_ref[...],
                preferred_element_type=jnp.float32) + coreb_ref[...], 0.0)
    pidx = pidx_ref[...]
    giota = lax.broadcasted_iota(jnp.int32, (tile, num_graphs), 1)
    onehot = (pidx == giota).astype(jnp.bfloat16)
    lhs = jnp.concatenate([onehot, par.astype(jnp.bfloat16)], axis=1)
    h1 = jnp.maximum(
        jnp.dot(lhs, precat_ref[...], preferred_element_type=jnp.float32), 0.0)
    h2 = jnp.maximum(
        jnp.dot(h1.astype(jnp.bfloat16), w2_ref[...],
                preferred_element_type=jnp.float32) + b2_ref[...], 0.0)
    out_ref[...] = (jnp.dot(h2.astype(jnp.bfloat16), w3_ref[...],
                            preferred_element_type=jnp.float32) + b3_ref[...])


def kernel(core_w, core_b, core_wg, core_bg, ep_w1_post, ep_w1_cur,
           ep_w1_glob, ep_b1, el_w1_cur, el_w1_post, el_w1_glob, el_b1,
           el_w2, el_b2, el_w3, el_b3, wout_el, wout_ep, bout,
           node_counts, node_offsets, graph_ids, node_features,
           partner_index_index, partner_index_values):
    n, f = node_features.shape
    e = EMBED_DIM
    b = node_counts.shape[0]
    p = partner_index_index.shape[0]
    hp = lax.Precision.HIGHEST

    node_post = jnp.maximum(node_features @ core_w + core_b, 0.0)
    seg_sum = jax.ops.segment_sum(node_post, graph_ids, num_segments=b)
    counts = jnp.maximum(node_counts.astype(jnp.float32), 1.0)[:, None]
    global_emb = (seg_sum / counts) @ core_wg + core_bg

    node_current = node_post[node_offsets[1:] - 1]
    ep_pre = (jnp.dot(node_current, ep_w1_cur, precision=hp)
              + jnp.dot(global_emb, ep_w1_glob, precision=hp) + ep_b1)
    el_pre = (jnp.dot(node_current, el_w1_cur, precision=hp)
              + jnp.dot(global_emb, el_w1_glob, precision=hp) + el_b1)

    corew_bf = core_w.astype(jnp.bfloat16)

    ep_wcat = jnp.concatenate([ep_pre, ep_w1_post], axis=0).astype(jnp.bfloat16)
    ep_w2 = wout_ep[:, NUM_EDGE_TYPES:NUM_EDGE_TYPES + 1].astype(jnp.bfloat16)
    ep_b2 = bout[:, NUM_EDGE_TYPES:NUM_EDGE_TYPES + 1]

    gids2 = graph_ids.astype(jnp.int32).reshape(n, 1)
    grid = (pl.cdiv(n, ROW_TILE),)
    partner = pl.pallas_call(
        functools.partial(_partner_kernel, num_graphs=b),
        out_shape=jax.ShapeDtypeStruct((n, 1), jnp.float32),
        grid=grid,
        in_specs=[
            pl.BlockSpec((ROW_TILE, 1), lambda i: (i, 0)),
            pl.BlockSpec((ROW_TILE, f), lambda i: (i, 0)),
            _pinned((b + e, e)),
            _pinned((f, e)),
            _pinned((1, e)),
            _pinned((e, 1)),
            _pinned((1, 1)),
        ],
        out_specs=pl.BlockSpec((ROW_TILE, 1), lambda i: (i, 0)),
        compiler_params=_compiler_params(),
    )(gids2, node_features, ep_wcat, corew_bf, core_b, ep_w2, ep_b2)
    edge_partner_logits = partner.reshape(n)

    par_nf = node_features[partner_index_values]
    el_precat = jnp.concatenate([el_pre, el_w1_post], axis=0).astype(jnp.bfloat16)
    pidx2 = partner_index_index.astype(jnp.int32).reshape(p, 1)

    label = pl.pallas_call(
        functools.partial(_label_kernel, num_graphs=b),
        out_shape=jax.ShapeDtypeStruct((p, NUM_EDGE_TYPES), jnp.float32),
        grid=(pl.cdiv(p, P_TILE),),
        in_specs=[
            pl.BlockSpec((P_TILE, 1), lambda i: (i, 0)),
            pl.BlockSpec((P_TILE, f), lambda i: (i, 0)),
            _pinned((b + e, e)),
            _pinned((f, e)),
            _pinned((1, e)),
            _pinned((e, e)),
            _pinned((1, e)),
            _pinned((e, NUM_EDGE_TYPES)),
            _pinned((1, NUM_EDGE_TYPES)),
        ],
        out_specs=pl.BlockSpec((P_TILE, NUM_EDGE_TYPES), lambda i: (i, 0)),
        compiler_params=_compiler_params(),
    )(pidx2, par_nf, el_precat, corew_bf, core_b,
      el_w2.astype(jnp.bfloat16), el_b2,
      el_w3.astype(jnp.bfloat16), el_b3)

    return {"edge_partner_logits": edge_partner_logits,
            "edge_label_logits": label}


# speedup vs baseline: 2.4462x; 1.8164x over previous
import functools

import jax
import jax.numpy as jnp
from jax import lax
from jax.experimental import pallas as pl
from jax.experimental.pallas import tpu as pltpu

EMBED_DIM = 32
NODE_FEAT_DIM = 16
NUM_EDGE_TYPES = 8
ROW_TILE = 2048
SUB = 128
WIN = 256
P_TILE = 2048


def _pinned(shape):
    return pl.BlockSpec(shape, lambda i: tuple(0 for _ in shape))


def _embed(nf_ref, corew_ref, coreb_ref):
    return jnp.maximum(
        jnp.dot(nf_ref[...].astype(jnp.bfloat16), corew_ref[...],
                preferred_element_type=jnp.float32) + coreb_ref[...], 0.0)


def _segsum_kernel(widx_ref, gidr_ref, nf_ref, corew_ref, coreb_ref,
                   out_ref, *, n_sub, steps_per_core):
    c = pl.program_id(0)
    i = pl.program_id(1)

    @pl.when(i == 0)
    def _():
        out_ref[...] = jnp.zeros_like(out_ref)

    x = _embed(nf_ref, corew_ref, coreb_ref).astype(jnp.bfloat16)
    gid_row = gidr_ref[0]
    base = (c * steps_per_core + i) * n_sub
    biota = lax.broadcasted_iota(jnp.int32, (WIN, SUB), 0).astype(jnp.bfloat16)
    for s in range(n_sub):
        wstart = pl.multiple_of(widx_ref[base + s], SUB)
        lgid = (gid_row[:, s * SUB:(s + 1) * SUB] - wstart).astype(jnp.bfloat16)
        oh_t = (biota == lgid).astype(jnp.bfloat16)
        part = jnp.dot(oh_t, x[s * SUB:(s + 1) * SUB, :],
                       preferred_element_type=jnp.float32)
        out_ref[0, pl.ds(wstart, WIN), :] += part


def _partner_kernel(widx_ref, gid_ref, nf_ref, eppre_ref, w1p_ref,
                    corew_ref, coreb_ref, w2_ref, b2_ref, out_ref, *, n_sub):
    i = pl.program_id(0)
    x = _embed(nf_ref, corew_ref, coreb_ref).astype(jnp.bfloat16)
    xw = jnp.dot(x, w1p_ref[...], preferred_element_type=jnp.float32)
    liota = lax.broadcasted_iota(jnp.int32, (SUB, WIN), 1).astype(jnp.bfloat16)
    hs = []
    for s in range(n_sub):
        wstart = pl.multiple_of(widx_ref[i * n_sub + s], SUB)
        lgid = (gid_ref[s * SUB:(s + 1) * SUB, :] - wstart).astype(jnp.bfloat16)
        oh = (lgid == liota).astype(jnp.bfloat16)
        pre = jnp.dot(oh, eppre_ref[pl.ds(wstart, WIN), :],
                      preferred_element_type=jnp.float32)
        h = jnp.maximum(xw[s * SUB:(s + 1) * SUB, :] + pre, 0.0)
        hs.append(h.astype(jnp.bfloat16))
    h_all = jnp.concatenate(hs, axis=0)
    out_ref[...] = (jnp.dot(h_all, w2_ref[...],
                            preferred_element_type=jnp.float32) + b2_ref[...])


def _label_kernel(pidx_ref, pnf_ref, precat_ref, corew_ref, coreb_ref,
                  w2_ref, b2_ref, w3_ref, b3_ref, out_ref, *, num_graphs):
    tile = pnf_ref.shape[0]
    par = _embed(pnf_ref, corew_ref, coreb_ref)
    pidx = pidx_ref[...]
    giota = lax.broadcasted_iota(jnp.int32, (tile, num_graphs), 1)
    onehot = (pidx == giota).astype(jnp.bfloat16)
    lhs = jnp.concatenate([onehot, par.astype(jnp.bfloat16)], axis=1)
    h1 = jnp.maximum(
        jnp.dot(lhs, precat_ref[...], preferred_element_type=jnp.float32), 0.0)
    h2 = jnp.maximum(
        jnp.dot(h1.astype(jnp.bfloat16), w2_ref[...],
                preferred_element_type=jnp.float32) + b2_ref[...], 0.0)
    out_ref[...] = (jnp.dot(h2.astype(jnp.bfloat16), w3_ref[...],
                            preferred_element_type=jnp.float32) + b3_ref[...])


def kernel(core_w, core_b, core_wg, core_bg, ep_w1_post, ep_w1_cur,
           ep_w1_glob, ep_b1, el_w1_cur, el_w1_post, el_w1_glob, el_b1,
           el_w2, el_b2, el_w3, el_b3, wout_el, wout_ep, bout,
           node_counts, node_offsets, graph_ids, node_features,
           partner_index_index, partner_index_values):
    n, f = node_features.shape
    e = EMBED_DIM
    b = node_counts.shape[0]
    p = partner_index_index.shape[0]
    hp = lax.Precision.HIGHEST

    corew_bf = core_w.astype(jnp.bfloat16)
    gids = graph_ids.astype(jnp.int32)

    w_idx = jnp.minimum((gids[::SUB] // SUB) * SUB, b - WIN)

    n_steps = n // ROW_TILE
    steps_per_core = n_steps // 2
    n_sub = ROW_TILE // SUB
    gid_rows = gids.reshape(n_steps, 1, ROW_TILE)
    seg_partial = pl.pallas_call(
        functools.partial(_segsum_kernel, n_sub=n_sub,
                          steps_per_core=steps_per_core),
        grid_spec=pltpu.PrefetchScalarGridSpec(
            num_scalar_prefetch=1,
            grid=(2, steps_per_core),
            in_specs=[
                pl.BlockSpec((1, 1, ROW_TILE),
                             lambda c, i, w: (c * steps_per_core + i, 0, 0)),
                pl.BlockSpec((ROW_TILE, f),
                             lambda c, i, w: (c * steps_per_core + i, 0)),
                pl.BlockSpec((f, e), lambda c, i, w: (0, 0)),
                pl.BlockSpec((1, e), lambda c, i, w: (0, 0)),
            ],
            out_specs=pl.BlockSpec((1, b, e), lambda c, i, w: (c, 0, 0)),
        ),
        out_shape=jax.ShapeDtypeStruct((2, b, e), jnp.float32),
        compiler_params=pltpu.CompilerParams(
            dimension_semantics=("parallel", "arbitrary"),
            vmem_limit_bytes=64 * 1024 * 1024,
        ),
    )(w_idx, gid_rows, node_features, corew_bf, core_b)
    seg_sum = seg_partial[0] + seg_partial[1]

    counts = jnp.maximum(node_counts.astype(jnp.float32), 1.0)[:, None]
    global_emb = (seg_sum / counts) @ core_wg + core_bg

    last_nf = node_features[node_offsets[1:] - 1]
    node_current = jnp.maximum(last_nf @ core_w + core_b, 0.0)
    ep_pre = (jnp.dot(node_current, ep_w1_cur, precision=hp)
              + jnp.dot(global_emb, ep_w1_glob, precision=hp) + ep_b1)
    el_pre = (jnp.dot(node_current, el_w1_cur, precision=hp)
              + jnp.dot(global_emb, el_w1_glob, precision=hp) + el_b1)

    ep_w2 = wout_ep[:, NUM_EDGE_TYPES:NUM_EDGE_TYPES + 1].astype(jnp.bfloat16)
    ep_b2 = bout[:, NUM_EDGE_TYPES:NUM_EDGE_TYPES + 1]

    partner = pl.pallas_call(
        functools.partial(_partner_kernel, n_sub=n_sub),
        grid_spec=pltpu.PrefetchScalarGridSpec(
            num_scalar_prefetch=1,
            grid=(n_steps,),
            in_specs=[
                pl.BlockSpec((ROW_TILE, 1), lambda i, w: (i, 0)),
                pl.BlockSpec((ROW_TILE, f), lambda i, w: (i, 0)),
                pl.BlockSpec((b, e), lambda i, w: (0, 0)),
                pl.BlockSpec((e, e), lambda i, w: (0, 0)),
                pl.BlockSpec((f, e), lambda i, w: (0, 0)),
                pl.BlockSpec((1, e), lambda i, w: (0, 0)),
                pl.BlockSpec((e, 1), lambda i, w: (0, 0)),
                pl.BlockSpec((1, 1), lambda i, w: (0, 0)),
            ],
            out_specs=pl.BlockSpec((ROW_TILE, 1), lambda i, w: (i, 0)),
        ),
        out_shape=jax.ShapeDtypeStruct((n, 1), jnp.float32),
        compiler_params=pltpu.CompilerParams(
            dimension_semantics=("parallel",),
            vmem_limit_bytes=64 * 1024 * 1024,
        ),
    )(w_idx, gids.reshape(n, 1), node_features,
      ep_pre.astype(jnp.bfloat16), ep_w1_post.astype(jnp.bfloat16),
      corew_bf, core_b, ep_w2, ep_b2)
    edge_partner_logits = partner.reshape(n)

    par_nf = node_features[partner_index_values]
    el_precat = jnp.concatenate([el_pre, el_w1_post], axis=0).astype(jnp.bfloat16)
    pidx2 = partner_index_index.astype(jnp.int32).reshape(p, 1)

    label = pl.pallas_call(
        functools.partial(_label_kernel, num_graphs=b),
        out_shape=jax.ShapeDtypeStruct((p, NUM_EDGE_TYPES), jnp.float32),
        grid=(pl.cdiv(p, P_TILE),),
        in_specs=[
            pl.BlockSpec((P_TILE, 1), lambda i: (i, 0)),
            pl.BlockSpec((P_TILE, f), lambda i: (i, 0)),
            _pinned((b + e, e)),
            _pinned((f, e)),
            _pinned((1, e)),
            _pinned((e, e)),
            _pinned((1, e)),
            _pinned((e, NUM_EDGE_TYPES)),
            _pinned((1, NUM_EDGE_TYPES)),
        ],
        out_specs=pl.BlockSpec((P_TILE, NUM_EDGE_TYPES), lambda i: (i, 0)),
        compiler_params=pltpu.CompilerParams(
            dimension_semantics=("parallel",),
            vmem_limit_bytes=64 * 1024 * 1024,
        ),
    )(pidx2, par_nf, el_precat, corew_bf, core_b,
      el_w2.astype(jnp.bfloat16), el_b2,
      el_w3.astype(jnp.bfloat16), el_b3)

    return {"edge_partner_logits": edge_partner_logits,
            "edge_label_logits": label}


# speedup vs baseline: 3.2721x; 1.3376x over previous
import functools

import jax
import jax.numpy as jnp
from jax import lax
from jax.experimental import pallas as pl
from jax.experimental.pallas import tpu as pltpu

EMBED_DIM = 32
NODE_FEAT_DIM = 16
NUM_EDGE_TYPES = 8
ROW_TILE = 4096
SUB = 128
WIN = 256
P_TILE = 2048


def _pinned(shape):
    return pl.BlockSpec(shape, lambda i: tuple(0 for _ in shape))


def _embed(nf_ref, corew_ref, coreb_ref):
    return jnp.maximum(
        jnp.dot(nf_ref[...].astype(jnp.bfloat16), corew_ref[...],
                preferred_element_type=jnp.float32) + coreb_ref[...], 0.0)


def _segsum_kernel(widx_ref, gidr_ref, nf_ref, corew_ref, coreb_ref,
                   out_ref, *, n_sub, steps_per_core):
    c = pl.program_id(0)
    i = pl.program_id(1)

    @pl.when(i == 0)
    def _():
        out_ref[...] = jnp.zeros_like(out_ref)

    x = _embed(nf_ref, corew_ref, coreb_ref).astype(jnp.bfloat16)
    gid_row = gidr_ref[0]
    base = (c * steps_per_core + i) * n_sub
    biota = lax.broadcasted_iota(jnp.int32, (WIN, SUB), 0).astype(jnp.bfloat16)
    for s in range(n_sub):
        wstart = pl.multiple_of(widx_ref[base + s], SUB)
        lgid = (gid_row[:, s * SUB:(s + 1) * SUB] - wstart).astype(jnp.bfloat16)
        oh_t = (biota == lgid).astype(jnp.bfloat16)
        part = jnp.dot(oh_t, x[s * SUB:(s + 1) * SUB, :],
                       preferred_element_type=jnp.float32)
        out_ref[0, pl.ds(wstart, WIN), :] += part


def _partner_kernel(widx_ref, gidr_ref, nf_ref, eppre_ref, w1p_ref,
                    corew_ref, coreb_ref, w2_ref, b2_ref, out_ref, *, n_sub):
    i = pl.program_id(0)
    x = _embed(nf_ref, corew_ref, coreb_ref).astype(jnp.bfloat16)
    xw = jnp.dot(x, w1p_ref[...], preferred_element_type=jnp.float32)
    gid_row = gidr_ref[0]
    biota = lax.broadcasted_iota(jnp.int32, (WIN, SUB), 0).astype(jnp.bfloat16)
    hs = []
    for s in range(n_sub):
        wstart = pl.multiple_of(widx_ref[i * n_sub + s], SUB)
        lgid = (gid_row[:, s * SUB:(s + 1) * SUB] - wstart).astype(jnp.bfloat16)
        oh_t = (biota == lgid).astype(jnp.bfloat16)
        pre = lax.dot_general(oh_t, eppre_ref[pl.ds(wstart, WIN), :],
                              (((0,), (0,)), ((), ())),
                              preferred_element_type=jnp.float32)
        h = jnp.maximum(xw[s * SUB:(s + 1) * SUB, :] + pre, 0.0)
        hs.append(h.astype(jnp.bfloat16))
    h_all = jnp.concatenate(hs, axis=0)
    out_ref[...] = (jnp.dot(h_all, w2_ref[...],
                            preferred_element_type=jnp.float32) + b2_ref[...])


def _label_kernel(pidx_ref, pnf_ref, precat_ref, corew_ref, coreb_ref,
                  w2_ref, b2_ref, w3_ref, b3_ref, out_ref, *, num_graphs):
    tile = pnf_ref.shape[0]
    par = _embed(pnf_ref, corew_ref, coreb_ref)
    pidx = pidx_ref[...]
    giota = lax.broadcasted_iota(jnp.int32, (tile, num_graphs), 1)
    onehot = (pidx == giota).astype(jnp.bfloat16)
    lhs = jnp.concatenate([onehot, par.astype(jnp.bfloat16)], axis=1)
    h1 = jnp.maximum(
        jnp.dot(lhs, precat_ref[...], preferred_element_type=jnp.float32), 0.0)
    h2 = jnp.maximum(
        jnp.dot(h1.astype(jnp.bfloat16), w2_ref[...],
                preferred_element_type=jnp.float32) + b2_ref[...], 0.0)
    out_ref[...] = (jnp.dot(h2.astype(jnp.bfloat16), w3_ref[...],
                            preferred_element_type=jnp.float32) + b3_ref[...])


def kernel(core_w, core_b, core_wg, core_bg, ep_w1_post, ep_w1_cur,
           ep_w1_glob, ep_b1, el_w1_cur, el_w1_post, el_w1_glob, el_b1,
           el_w2, el_b2, el_w3, el_b3, wout_el, wout_ep, bout,
           node_counts, node_offsets, graph_ids, node_features,
           partner_index_index, partner_index_values):
    n, f = node_features.shape
    e = EMBED_DIM
    b = node_counts.shape[0]
    p = partner_index_index.shape[0]
    hp = lax.Precision.HIGHEST

    corew_bf = core_w.astype(jnp.bfloat16)
    gids = graph_ids.astype(jnp.int32)

    w_idx = jnp.minimum((gids[::SUB] // SUB) * SUB, b - WIN)

    n_steps = n // ROW_TILE
    steps_per_core = n_steps // 2
    n_sub = ROW_TILE // SUB
    gid_rows = gids.reshape(n_steps, 1, ROW_TILE)
    seg_partial = pl.pallas_call(
        functools.partial(_segsum_kernel, n_sub=n_sub,
                          steps_per_core=steps_per_core),
        grid_spec=pltpu.PrefetchScalarGridSpec(
            num_scalar_prefetch=1,
            grid=(2, steps_per_core),
            in_specs=[
                pl.BlockSpec((1, 1, ROW_TILE),
                             lambda c, i, w: (c * steps_per_core + i, 0, 0)),
                pl.BlockSpec((ROW_TILE, f),
                             lambda c, i, w: (c * steps_per_core + i, 0)),
                pl.BlockSpec((f, e), lambda c, i, w: (0, 0)),
                pl.BlockSpec((1, e), lambda c, i, w: (0, 0)),
            ],
            out_specs=pl.BlockSpec((1, b, e), lambda c, i, w: (c, 0, 0)),
        ),
        out_shape=jax.ShapeDtypeStruct((2, b, e), jnp.float32),
        compiler_params=pltpu.CompilerParams(
            dimension_semantics=("parallel", "arbitrary"),
            vmem_limit_bytes=64 * 1024 * 1024,
        ),
    )(w_idx, gid_rows, node_features, corew_bf, core_b)
    seg_sum = seg_partial[0] + seg_partial[1]

    counts = jnp.maximum(node_counts.astype(jnp.float32), 1.0)[:, None]
    global_emb = (seg_sum / counts) @ core_wg + core_bg

    last_nf = node_features[node_offsets[1:] - 1]
    node_current = jnp.maximum(last_nf @ core_w + core_b, 0.0)
    ep_pre = (jnp.dot(node_current, ep_w1_cur, precision=hp)
              + jnp.dot(global_emb, ep_w1_glob, precision=hp) + ep_b1)
    el_pre = (jnp.dot(node_current, el_w1_cur, precision=hp)
              + jnp.dot(global_emb, el_w1_glob, precision=hp) + el_b1)

    ep_w2 = wout_ep[:, NUM_EDGE_TYPES:NUM_EDGE_TYPES + 1].astype(jnp.bfloat16)
    ep_b2 = bout[:, NUM_EDGE_TYPES:NUM_EDGE_TYPES + 1]

    partner = pl.pallas_call(
        functools.partial(_partner_kernel, n_sub=n_sub),
        grid_spec=pltpu.PrefetchScalarGridSpec(
            num_scalar_prefetch=1,
            grid=(n_steps,),
            in_specs=[
                pl.BlockSpec((1, 1, ROW_TILE), lambda i, w: (i, 0, 0)),
                pl.BlockSpec((ROW_TILE, f), lambda i, w: (i, 0)),
                pl.BlockSpec((b, e), lambda i, w: (0, 0)),
                pl.BlockSpec((e, e), lambda i, w: (0, 0)),
                pl.BlockSpec((f, e), lambda i, w: (0, 0)),
                pl.BlockSpec((1, e), lambda i, w: (0, 0)),
                pl.BlockSpec((e, 1), lambda i, w: (0, 0)),
                pl.BlockSpec((1, 1), lambda i, w: (0, 0)),
            ],
            out_specs=pl.BlockSpec((ROW_TILE, 1), lambda i, w: (i, 0)),
        ),
        out_shape=jax.ShapeDtypeStruct((n, 1), jnp.float32),
        compiler_params=pltpu.CompilerParams(
            dimension_semantics=("parallel",),
            vmem_limit_bytes=64 * 1024 * 1024,
        ),
    )(w_idx, gid_rows, node_features,
      ep_pre.astype(jnp.bfloat16), ep_w1_post.astype(jnp.bfloat16),
      corew_bf, core_b, ep_w2, ep_b2)
    edge_partner_logits = partner.reshape(n)

    par_nf = node_features[partner_index_values]
    el_precat = jnp.concatenate([el_pre, el_w1_post], axis=0).astype(jnp.bfloat16)
    pidx2 = partner_index_index.astype(jnp.int32).reshape(p, 1)

    label = pl.pallas_call(
        functools.partial(_label_kernel, num_graphs=b),
        out_shape=jax.ShapeDtypeStruct((p, NUM_EDGE_TYPES), jnp.float32),
        grid=(pl.cdiv(p, P_TILE),),
        in_specs=[
            pl.BlockSpec((P_TILE, 1), lambda i: (i, 0)),
            pl.BlockSpec((P_TILE, f), lambda i: (i, 0)),
            _pinned((b + e, e)),
            _pinned((f, e)),
            _pinned((1, e)),
            _pinned((e, e)),
            _pinned((1, e)),
            _pinned((e, NUM_EDGE_TYPES)),
            _pinned((1, NUM_EDGE_TYPES)),
        ],
        out_specs=pl.BlockSpec((P_TILE, NUM_EDGE_TYPES), lambda i: (i, 0)),
        compiler_params=pltpu.CompilerParams(
            dimension_semantics=("parallel",),
            vmem_limit_bytes=64 * 1024 * 1024,
        ),
    )(pidx2, par_nf, el_precat, corew_bf, core_b,
      el_w2.astype(jnp.bfloat16), el_b2,
      el_w3.astype(jnp.bfloat16), el_b3)

    return {"edge_partner_logits": edge_partner_logits,
            "edge_label_logits": label}


# speedup vs baseline: 3.7135x; 1.1349x over previous
import functools

import jax
import jax.numpy as jnp
from jax import lax
from jax.experimental import pallas as pl
from jax.experimental.pallas import tpu as pltpu

EMBED_DIM = 32
NODE_FEAT_DIM = 16
NUM_EDGE_TYPES = 8
ROW_TILE = 4096
SUB = 128
WIN = 256
P_TILE = 2048


def _pinned(shape):
    return pl.BlockSpec(shape, lambda i: tuple(0 for _ in shape))


def _embed(nf_ref, corew_ref, coreb_ref):
    return jnp.maximum(
        jnp.dot(nf_ref[...].astype(jnp.bfloat16), corew_ref[...],
                preferred_element_type=jnp.float32) + coreb_ref[...], 0.0)


def _segsum_kernel(widx_ref, gidr_ref, nf_ref, corew_ref, coreb_ref,
                   out_ref, *, n_sub, steps_per_core):
    c = pl.program_id(0)
    i = pl.program_id(1)

    @pl.when(i == 0)
    def _():
        out_ref[...] = jnp.zeros_like(out_ref)

    x = _embed(nf_ref, corew_ref, coreb_ref).astype(jnp.bfloat16)
    gid_row = gidr_ref[0]
    base = (c * steps_per_core + i) * n_sub
    biota = lax.broadcasted_iota(jnp.int32, (WIN, SUB), 0).astype(jnp.bfloat16)
    for s in range(n_sub):
        wstart = pl.multiple_of(widx_ref[base + s], SUB)
        lgid = (gid_row[:, s * SUB:(s + 1) * SUB] - wstart).astype(jnp.bfloat16)
        oh_t = (biota == lgid).astype(jnp.bfloat16)
        part = jnp.dot(oh_t, x[s * SUB:(s + 1) * SUB, :],
                       preferred_element_type=jnp.float32)
        out_ref[0, pl.ds(wstart, WIN), :] += part


def _partner_kernel(widx_ref, gidr_ref, nf_ref, eppre_ref, w1p_ref,
                    corew_ref, coreb_ref, w2_ref, b2_ref, out_ref, *, n_sub):
    i = pl.program_id(0)
    x = _embed(nf_ref, corew_ref, coreb_ref).astype(jnp.bfloat16)
    xw = jnp.dot(x, w1p_ref[...], preferred_element_type=jnp.float32)
    gid_row = gidr_ref[0]
    biota = lax.broadcasted_iota(jnp.int32, (WIN, SUB), 0).astype(jnp.bfloat16)
    hs = []
    for s in range(n_sub):
        wstart = pl.multiple_of(widx_ref[i * n_sub + s], SUB)
        lgid = (gid_row[:, s * SUB:(s + 1) * SUB] - wstart).astype(jnp.bfloat16)
        oh_t = (biota == lgid).astype(jnp.bfloat16)
        pre = lax.dot_general(oh_t, eppre_ref[pl.ds(wstart, WIN), :],
                              (((0,), (0,)), ((), ())),
                              preferred_element_type=jnp.float32)
        h = jnp.maximum(xw[s * SUB:(s + 1) * SUB, :] + pre, 0.0)
        hs.append(h.astype(jnp.bfloat16))
    h_all = jnp.concatenate(hs, axis=0)
    row = lax.dot_general(w2_ref[...], h_all, (((0,), (1,)), ((), ())),
                          preferred_element_type=jnp.float32)
    out_ref[0] = row + b2_ref[...]


def _label_kernel(pidx_ref, pnf_ref, precat_ref, corew_ref, coreb_ref,
                  w2_ref, b2_ref, w3_ref, b3_ref, out_ref, *, num_graphs):
    tile = pnf_ref.shape[0]
    par = _embed(pnf_ref, corew_ref, coreb_ref)
    pidx = pidx_ref[...]
    giota = lax.broadcasted_iota(jnp.int32, (tile, num_graphs), 1)
    onehot = (pidx == giota).astype(jnp.bfloat16)
    lhs = jnp.concatenate([onehot, par.astype(jnp.bfloat16)], axis=1)
    h1 = jnp.maximum(
        jnp.dot(lhs, precat_ref[...], preferred_element_type=jnp.float32), 0.0)
    h2 = jnp.maximum(
        jnp.dot(h1.astype(jnp.bfloat16), w2_ref[...],
                preferred_element_type=jnp.float32) + b2_ref[...], 0.0)
    out_ref[...] = (jnp.dot(h2.astype(jnp.bfloat16), w3_ref[...],
                            preferred_element_type=jnp.float32) + b3_ref[...])


def kernel(core_w, core_b, core_wg, core_bg, ep_w1_post, ep_w1_cur,
           ep_w1_glob, ep_b1, el_w1_cur, el_w1_post, el_w1_glob, el_b1,
           el_w2, el_b2, el_w3, el_b3, wout_el, wout_ep, bout,
           node_counts, node_offsets, graph_ids, node_features,
           partner_index_index, partner_index_values):
    n, f = node_features.shape
    e = EMBED_DIM
    b = node_counts.shape[0]
    p = partner_index_index.shape[0]
    hp = lax.Precision.HIGHEST

    corew_bf = core_w.astype(jnp.bfloat16)
    gids = graph_ids.astype(jnp.int32)

    w_idx = jnp.minimum((gids[::SUB] // SUB) * SUB, b - WIN)

    n_steps = n // ROW_TILE
    steps_per_core = n_steps // 2
    n_sub = ROW_TILE // SUB
    gid_rows = gids.reshape(n_steps, 1, ROW_TILE)
    seg_partial = pl.pallas_call(
        functools.partial(_segsum_kernel, n_sub=n_sub,
                          steps_per_core=steps_per_core),
        grid_spec=pltpu.PrefetchScalarGridSpec(
            num_scalar_prefetch=1,
            grid=(2, steps_per_core),
            in_specs=[
                pl.BlockSpec((1, 1, ROW_TILE),
                             lambda c, i, w: (c * steps_per_core + i, 0, 0)),
                pl.BlockSpec((ROW_TILE, f),
                             lambda c, i, w: (c * steps_per_core + i, 0)),
                pl.BlockSpec((f, e), lambda c, i, w: (0, 0)),
                pl.BlockSpec((1, e), lambda c, i, w: (0, 0)),
            ],
            out_specs=pl.BlockSpec((1, b, e), lambda c, i, w: (c, 0, 0)),
        ),
        out_shape=jax.ShapeDtypeStruct((2, b, e), jnp.float32),
        compiler_params=pltpu.CompilerParams(
            dimension_semantics=("parallel", "arbitrary"),
            vmem_limit_bytes=64 * 1024 * 1024,
        ),
    )(w_idx, gid_rows, node_features, corew_bf, core_b)
    seg_sum = seg_partial[0] + seg_partial[1]

    counts = jnp.maximum(node_counts.astype(jnp.float32), 1.0)[:, None]
    global_emb = (seg_sum / counts) @ core_wg + core_bg

    last_nf = node_features[node_offsets[1:] - 1]
    node_current = jnp.maximum(last_nf @ core_w + core_b, 0.0)
    ep_pre = (jnp.dot(node_current, ep_w1_cur, precision=hp)
              + jnp.dot(global_emb, ep_w1_glob, precision=hp) + ep_b1)
    el_pre = (jnp.dot(node_current, el_w1_cur, precision=hp)
              + jnp.dot(global_emb, el_w1_glob, precision=hp) + el_b1)

    ep_w2 = wout_ep[:, NUM_EDGE_TYPES:NUM_EDGE_TYPES + 1].astype(jnp.bfloat16)
    ep_b2 = bout[:, NUM_EDGE_TYPES:NUM_EDGE_TYPES + 1]

    partner = pl.pallas_call(
        functools.partial(_partner_kernel, n_sub=n_sub),
        grid_spec=pltpu.PrefetchScalarGridSpec(
            num_scalar_prefetch=1,
            grid=(n_steps,),
            in_specs=[
                pl.BlockSpec((1, 1, ROW_TILE), lambda i, w: (i, 0, 0)),
                pl.BlockSpec((ROW_TILE, f), lambda i, w: (i, 0)),
                pl.BlockSpec((b, e), lambda i, w: (0, 0)),
                pl.BlockSpec((e, e), lambda i, w: (0, 0)),
                pl.BlockSpec((f, e), lambda i, w: (0, 0)),
                pl.BlockSpec((1, e), lambda i, w: (0, 0)),
                pl.BlockSpec((e, 1), lambda i, w: (0, 0)),
                pl.BlockSpec((1, 1), lambda i, w: (0, 0)),
            ],
            out_specs=pl.BlockSpec((1, 1, ROW_TILE), lambda i, w: (i, 0, 0)),
        ),
        out_shape=jax.ShapeDtypeStruct((n_steps, 1, ROW_TILE), jnp.float32),
        compiler_params=pltpu.CompilerParams(
            dimension_semantics=("parallel",),
            vmem_limit_bytes=64 * 1024 * 1024,
        ),
    )(w_idx, gid_rows, node_features,
      ep_pre.astype(jnp.bfloat16), ep_w1_post.astype(jnp.bfloat16),
      corew_bf, core_b, ep_w2, ep_b2)
    edge_partner_logits = partner.reshape(n)

    par_nf = node_features[partner_index_values]
    el_precat = jnp.concatenate([el_pre, el_w1_post], axis=0).astype(jnp.bfloat16)
    pidx2 = partner_index_index.astype(jnp.int32).reshape(p, 1)

    label = pl.pallas_call(
        functools.partial(_label_kernel, num_graphs=b),
        out_shape=jax.ShapeDtypeStruct((p, NUM_EDGE_TYPES), jnp.float32),
        grid=(pl.cdiv(p, P_TILE),),
        in_specs=[
            pl.BlockSpec((P_TILE, 1), lambda i: (i, 0)),
            pl.BlockSpec((P_TILE, f), lambda i: (i, 0)),
            _pinned((b + e, e)),
            _pinned((f, e)),
            _pinned((1, e)),
            _pinned((e, e)),
            _pinned((1, e)),
            _pinned((e, NUM_EDGE_TYPES)),
            _pinned((1, NUM_EDGE_TYPES)),
        ],
        out_specs=pl.BlockSpec((P_TILE, NUM_EDGE_TYPES), lambda i: (i, 0)),
        compiler_params=pltpu.CompilerParams(
            dimension_semantics=("parallel",),
            vmem_limit_bytes=64 * 1024 * 1024,
        ),
    )(pidx2, par_nf, el_precat, corew_bf, core_b,
      el_w2.astype(jnp.bfloat16), el_b2,
      el_w3.astype(jnp.bfloat16), el_b3)

    return {"edge_partner_logits": edge_partner_logits,
            "edge_label_logits": label}


# speedup vs baseline: 5.5259x; 1.4881x over previous
import functools

import jax
import jax.numpy as jnp
from jax import lax
from jax.experimental import pallas as pl
from jax.experimental.pallas import tpu as pltpu

EMBED_DIM = 32
NODE_FEAT_DIM = 16
NUM_EDGE_TYPES = 8
ROW_TILE = 4096
SUB = 128
WIN = 256
P_TILE = 2048


def _pinned(shape):
    return pl.BlockSpec(shape, lambda i: tuple(0 for _ in shape))


def _onehot_w(gid_row, wstart, biota, s):
    lgid = (gid_row[:, s * SUB:(s + 1) * SUB] - wstart).astype(jnp.bfloat16)
    return (biota == lgid).astype(jnp.bfloat16)


def _segsum_kernel(widx_ref, gidr_ref, xt_ref, out_ref,
                   *, n_sub, steps_per_core):
    c = pl.program_id(0)
    i = pl.program_id(1)

    @pl.when(i == 0)
    def _():
        out_ref[...] = jnp.zeros_like(out_ref)

    xt = xt_ref[...]
    gid_row = gidr_ref[0]
    base = (c * steps_per_core + i) * n_sub
    biota = lax.broadcasted_iota(jnp.int32, (WIN, SUB), 0).astype(jnp.bfloat16)
    for s in range(n_sub):
        wstart = pl.multiple_of(widx_ref[base + s], SUB)
        oh_w = _onehot_w(gid_row, wstart, biota, s)
        part = lax.dot_general(xt[:, s * SUB:(s + 1) * SUB], oh_w,
                               (((1,), (1,)), ((), ())),
                               preferred_element_type=jnp.float32)
        out_ref[0, :, pl.ds(wstart, WIN)] += part


def _partner_kernel(widx_ref, gidr_ref, xt_ref, eppret_ref, w1pt_ref,
                    w2_ref, b2_ref, out_ref, *, n_sub):
    i = pl.program_id(0)
    xt = xt_ref[...]
    xw = jnp.dot(w1pt_ref[...], xt, preferred_element_type=jnp.float32)
    gid_row = gidr_ref[0]
    biota = lax.broadcasted_iota(jnp.int32, (WIN, SUB), 0).astype(jnp.bfloat16)
    pres = []
    for s in range(n_sub):
        wstart = pl.multiple_of(widx_ref[i * n_sub + s], SUB)
        oh_w = _onehot_w(gid_row, wstart, biota, s)
        pres.append(jnp.dot(eppret_ref[:, pl.ds(wstart, WIN)], oh_w,
                            preferred_element_type=jnp.float32))
    pre_t = jnp.concatenate(pres, axis=1)
    h_t = jnp.maximum(xw + pre_t, 0.0).astype(jnp.bfloat16)
    row = lax.dot_general(w2_ref[...], h_t, (((0,), (0,)), ((), ())),
                          preferred_element_type=jnp.float32)
    out_ref[0] = row + b2_ref[...]


def _label_kernel(pidx_ref, pnf_ref, precat_ref, corew_ref, coreb_ref,
                  w2_ref, b2_ref, w3_ref, b3_ref, out_ref, *, num_graphs):
    tile = pnf_ref.shape[0]
    par = jnp.maximum(
        jnp.dot(pnf_ref[...].astype(jnp.bfloat16), corew_ref[...],
                preferred_element_type=jnp.float32) + coreb_ref[...], 0.0)
    pidx = pidx_ref[...]
    giota = lax.broadcasted_iota(jnp.int32, (tile, num_graphs), 1)
    onehot = (pidx == giota).astype(jnp.bfloat16)
    lhs = jnp.concatenate([onehot, par.astype(jnp.bfloat16)], axis=1)
    h1 = jnp.maximum(
        jnp.dot(lhs, precat_ref[...], preferred_element_type=jnp.float32), 0.0)
    h2 = jnp.maximum(
        jnp.dot(h1.astype(jnp.bfloat16), w2_ref[...],
                preferred_element_type=jnp.float32) + b2_ref[...], 0.0)
    out_ref[...] = (jnp.dot(h2.astype(jnp.bfloat16), w3_ref[...],
                            preferred_element_type=jnp.float32) + b3_ref[...])


def kernel(core_w, core_b, core_wg, core_bg, ep_w1_post, ep_w1_cur,
           ep_w1_glob, ep_b1, el_w1_cur, el_w1_post, el_w1_glob, el_b1,
           el_w2, el_b2, el_w3, el_b3, wout_el, wout_ep, bout,
           node_counts, node_offsets, graph_ids, node_features,
           partner_index_index, partner_index_values):
    n, f = node_features.shape
    e = EMBED_DIM
    b = node_counts.shape[0]
    p = partner_index_index.shape[0]
    hp = lax.Precision.HIGHEST

    gids = graph_ids.astype(jnp.int32)

    x = jnp.maximum(node_features @ core_w + core_b, 0.0)
    xt_bf = x.T.astype(jnp.bfloat16)

    w_idx = jnp.minimum((gids[::SUB] // SUB) * SUB, b - WIN)

    n_steps = n // ROW_TILE
    steps_per_core = n_steps // 2
    n_sub = ROW_TILE // SUB
    gid_rows = gids.reshape(n_steps, 1, ROW_TILE)

    seg_partial = pl.pallas_call(
        functools.partial(_segsum_kernel, n_sub=n_sub,
                          steps_per_core=steps_per_core),
        grid_spec=pltpu.PrefetchScalarGridSpec(
            num_scalar_prefetch=1,
            grid=(2, steps_per_core),
            in_specs=[
                pl.BlockSpec((1, 1, ROW_TILE),
                             lambda c, i, w: (c * steps_per_core + i, 0, 0)),
                pl.BlockSpec((e, ROW_TILE),
                             lambda c, i, w: (0, c * steps_per_core + i)),
            ],
            out_specs=pl.BlockSpec((1, e, b), lambda c, i, w: (c, 0, 0)),
        ),
        out_shape=jax.ShapeDtypeStruct((2, e, b), jnp.float32),
        compiler_params=pltpu.CompilerParams(
            dimension_semantics=("parallel", "arbitrary"),
            vmem_limit_bytes=64 * 1024 * 1024,
        ),
    )(w_idx, gid_rows, xt_bf)
    seg_sum = (seg_partial[0] + seg_partial[1]).T

    counts = jnp.maximum(node_counts.astype(jnp.float32), 1.0)[:, None]
    global_emb = (seg_sum / counts) @ core_wg + core_bg

    node_current = x[node_offsets[1:] - 1]
    ep_pre = (jnp.dot(node_current, ep_w1_cur, precision=hp)
              + jnp.dot(global_emb, ep_w1_glob, precision=hp) + ep_b1)
    el_pre = (jnp.dot(node_current, el_w1_cur, precision=hp)
              + jnp.dot(global_emb, el_w1_glob, precision=hp) + el_b1)

    ep_w2 = wout_ep[:, NUM_EDGE_TYPES:NUM_EDGE_TYPES + 1].astype(jnp.bfloat16)
    ep_b2 = bout[:, NUM_EDGE_TYPES:NUM_EDGE_TYPES + 1]

    partner = pl.pallas_call(
        functools.partial(_partner_kernel, n_sub=n_sub),
        grid_spec=pltpu.PrefetchScalarGridSpec(
            num_scalar_prefetch=1,
            grid=(n_steps,),
            in_specs=[
                pl.BlockSpec((1, 1, ROW_TILE), lambda i, w: (i, 0, 0)),
                pl.BlockSpec((e, ROW_TILE), lambda i, w: (0, i)),
                pl.BlockSpec((e, b), lambda i, w: (0, 0)),
                pl.BlockSpec((e, e), lambda i, w: (0, 0)),
                pl.BlockSpec((e, 1), lambda i, w: (0, 0)),
                pl.BlockSpec((1, 1), lambda i, w: (0, 0)),
            ],
            out_specs=pl.BlockSpec((1, 1, ROW_TILE), lambda i, w: (i, 0, 0)),
        ),
        out_shape=jax.ShapeDtypeStruct((n_steps, 1, ROW_TILE), jnp.float32),
        compiler_params=pltpu.CompilerParams(
            dimension_semantics=("parallel",),
            vmem_limit_bytes=64 * 1024 * 1024,
        ),
    )(w_idx, gid_rows, xt_bf,
      ep_pre.T.astype(jnp.bfloat16), ep_w1_post.T.astype(jnp.bfloat16),
      ep_w2, ep_b2)
    edge_partner_logits = partner.reshape(n)

    par_nf = node_features[partner_index_values]
    el_precat = jnp.concatenate([el_pre, el_w1_post], axis=0).astype(jnp.bfloat16)
    pidx2 = partner_index_index.astype(jnp.int32).reshape(p, 1)

    label = pl.pallas_call(
        functools.partial(_label_kernel, num_graphs=b),
        out_shape=jax.ShapeDtypeStruct((p, NUM_EDGE_TYPES), jnp.float32),
        grid=(pl.cdiv(p, P_TILE),),
        in_specs=[
            pl.BlockSpec((P_TILE, 1), lambda i: (i, 0)),
            pl.BlockSpec((P_TILE, f), lambda i: (i, 0)),
            _pinned((b + e, e)),
            _pinned((f, e)),
            _pinned((1, e)),
            _pinned((e, e)),
            _pinned((1, e)),
            _pinned((e, NUM_EDGE_TYPES)),
            _pinned((1, NUM_EDGE_TYPES)),
        ],
        out_specs=pl.BlockSpec((P_TILE, NUM_EDGE_TYPES), lambda i: (i, 0)),
        compiler_params=pltpu.CompilerParams(
            dimension_semantics=("parallel",),
            vmem_limit_bytes=64 * 1024 * 1024,
        ),
    )(pidx2, par_nf, el_precat, core_w.astype(jnp.bfloat16), core_b,
      el_w2.astype(jnp.bfloat16), el_b2,
      el_w3.astype(jnp.bfloat16), el_b3)

    return {"edge_partner_logits": edge_partner_logits,
            "edge_label_logits": label}


# speedup vs baseline: 6.0215x; 1.0897x over previous
import functools

import jax
import jax.numpy as jnp
from jax import lax
from jax.experimental import pallas as pl
from jax.experimental.pallas import tpu as pltpu

EMBED_DIM = 32
NODE_FEAT_DIM = 16
NUM_EDGE_TYPES = 8
ROW_TILE = 4096
SUB = 128
WIN = 256
P_TILE = 2048


def _pinned(shape):
    return pl.BlockSpec(shape, lambda i: tuple(0 for _ in shape))


def _onehot_w(gid_row, wstart, biota, s):
    lgid = (gid_row[:, s * SUB:(s + 1) * SUB] - wstart).astype(jnp.bfloat16)
    return (biota == lgid).astype(jnp.bfloat16)


def _segsum_kernel(widx_ref, gidr_ref, xt_ref, out_ref,
                   *, n_sub, steps_per_core):
    c = pl.program_id(0)
    i = pl.program_id(1)

    @pl.when(i == 0)
    def _():
        out_ref[...] = jnp.zeros_like(out_ref)

    xt = xt_ref[...]
    gid_row = gidr_ref[0]
    base = (c * steps_per_core + i) * n_sub
    biota = lax.broadcasted_iota(jnp.int32, (WIN, SUB), 0).astype(jnp.bfloat16)
    for s in range(n_sub):
        wstart = pl.multiple_of(widx_ref[base + s], SUB)
        oh_w = _onehot_w(gid_row, wstart, biota, s)
        part = lax.dot_general(xt[:, s * SUB:(s + 1) * SUB], oh_w,
                               (((1,), (1,)), ((), ())),
                               preferred_element_type=jnp.float32)
        out_ref[0, :, pl.ds(wstart, WIN)] += part


def _partner_kernel(widx_ref, gidr_ref, xt_ref, eppret_ref, w1pt_ref,
                    w2_ref, b2_ref, out_ref, *, n_sub):
    i = pl.program_id(0)
    xt = xt_ref[...]
    xw = jnp.dot(w1pt_ref[...], xt, preferred_element_type=jnp.float32)
    gid_row = gidr_ref[0]
    biota = lax.broadcasted_iota(jnp.int32, (WIN, SUB), 0).astype(jnp.bfloat16)
    pres = []
    for s in range(n_sub):
        wstart = pl.multiple_of(widx_ref[i * n_sub + s], SUB)
        oh_w = _onehot_w(gid_row, wstart, biota, s)
        pres.append(jnp.dot(eppret_ref[:, pl.ds(wstart, WIN)], oh_w,
                            preferred_element_type=jnp.float32))
    pre_t = jnp.concatenate(pres, axis=1)
    h_t = jnp.maximum(xw + pre_t, 0.0).astype(jnp.bfloat16)
    row = lax.dot_general(w2_ref[...], h_t, (((0,), (0,)), ((), ())),
                          preferred_element_type=jnp.float32)
    out_ref[0] = row + b2_ref[...]


def _label_kernel(pidx_ref, pnf_ref, precat_ref, corew_ref, coreb_ref,
                  w2_ref, b2_ref, w3_ref, b3_ref, out_ref, *, num_graphs):
    tile = pnf_ref.shape[0]
    par = jnp.maximum(
        jnp.dot(pnf_ref[...].astype(jnp.bfloat16), corew_ref[...],
                preferred_element_type=jnp.float32) + coreb_ref[...], 0.0)
    pidx = pidx_ref[...]
    giota = lax.broadcasted_iota(jnp.int32, (tile, num_graphs), 1)
    onehot = (pidx == giota).astype(jnp.bfloat16)
    lhs = jnp.concatenate([onehot, par.astype(jnp.bfloat16)], axis=1)
    h1 = jnp.maximum(
        jnp.dot(lhs, precat_ref[...], preferred_element_type=jnp.float32), 0.0)
    h2 = jnp.maximum(
        jnp.dot(h1.astype(jnp.bfloat16), w2_ref[...],
                preferred_element_type=jnp.float32) + b2_ref[...], 0.0)
    out_ref[...] = (jnp.dot(h2.astype(jnp.bfloat16), w3_ref[...],
                            preferred_element_type=jnp.float32) + b3_ref[...])


def kernel(core_w, core_b, core_wg, core_bg, ep_w1_post, ep_w1_cur,
           ep_w1_glob, ep_b1, el_w1_cur, el_w1_post, el_w1_glob, el_b1,
           el_w2, el_b2, el_w3, el_b3, wout_el, wout_ep, bout,
           node_counts, node_offsets, graph_ids, node_features,
           partner_index_index, partner_index_values):
    n, f = node_features.shape
    e = EMBED_DIM
    b = node_counts.shape[0]
    p = partner_index_index.shape[0]
    hp = lax.Precision.HIGHEST

    gids = graph_ids.astype(jnp.int32)

    x = jnp.maximum(node_features @ core_w + core_b, 0.0)
    xt_bf = x.T.astype(jnp.bfloat16)
    del x

    w_idx = jnp.minimum((gids[::SUB] // SUB) * SUB, b - WIN)

    n_steps = n // ROW_TILE
    steps_per_core = n_steps // 2
    n_sub = ROW_TILE // SUB
    gid_rows = gids.reshape(n_steps, 1, ROW_TILE)

    seg_partial = pl.pallas_call(
        functools.partial(_segsum_kernel, n_sub=n_sub,
                          steps_per_core=steps_per_core),
        grid_spec=pltpu.PrefetchScalarGridSpec(
            num_scalar_prefetch=1,
            grid=(2, steps_per_core),
            in_specs=[
                pl.BlockSpec((1, 1, ROW_TILE),
                             lambda c, i, w: (c * steps_per_core + i, 0, 0)),
                pl.BlockSpec((e, ROW_TILE),
                             lambda c, i, w: (0, c * steps_per_core + i)),
            ],
            out_specs=pl.BlockSpec((1, e, b), lambda c, i, w: (c, 0, 0)),
        ),
        out_shape=jax.ShapeDtypeStruct((2, e, b), jnp.float32),
        compiler_params=pltpu.CompilerParams(
            dimension_semantics=("parallel", "arbitrary"),
            vmem_limit_bytes=64 * 1024 * 1024,
        ),
    )(w_idx, gid_rows, xt_bf)
    seg_sum = (seg_partial[0] + seg_partial[1]).T

    counts = jnp.maximum(node_counts.astype(jnp.float32), 1.0)[:, None]
    global_emb = (seg_sum / counts) @ core_wg + core_bg

    last_nf = node_features[node_offsets[1:] - 1]
    node_current = jnp.maximum(last_nf @ core_w + core_b, 0.0)
    ep_pre = (jnp.dot(node_current, ep_w1_cur, precision=hp)
              + jnp.dot(global_emb, ep_w1_glob, precision=hp) + ep_b1)
    el_pre = (jnp.dot(node_current, el_w1_cur, precision=hp)
              + jnp.dot(global_emb, el_w1_glob, precision=hp) + el_b1)

    ep_w2 = wout_ep[:, NUM_EDGE_TYPES:NUM_EDGE_TYPES + 1].astype(jnp.bfloat16)
    ep_b2 = bout[:, NUM_EDGE_TYPES:NUM_EDGE_TYPES + 1]

    partner = pl.pallas_call(
        functools.partial(_partner_kernel, n_sub=n_sub),
        grid_spec=pltpu.PrefetchScalarGridSpec(
            num_scalar_prefetch=1,
            grid=(n_steps,),
            in_specs=[
                pl.BlockSpec((1, 1, ROW_TILE), lambda i, w: (i, 0, 0)),
                pl.BlockSpec((e, ROW_TILE), lambda i, w: (0, i)),
                pl.BlockSpec((e, b), lambda i, w: (0, 0)),
                pl.BlockSpec((e, e), lambda i, w: (0, 0)),
                pl.BlockSpec((e, 1), lambda i, w: (0, 0)),
                pl.BlockSpec((1, 1), lambda i, w: (0, 0)),
            ],
            out_specs=pl.BlockSpec((1, 1, ROW_TILE), lambda i, w: (i, 0, 0)),
        ),
        out_shape=jax.ShapeDtypeStruct((n_steps, 1, ROW_TILE), jnp.float32),
        compiler_params=pltpu.CompilerParams(
            dimension_semantics=("parallel",),
            vmem_limit_bytes=64 * 1024 * 1024,
        ),
    )(w_idx, gid_rows, xt_bf,
      ep_pre.T.astype(jnp.bfloat16), ep_w1_post.T.astype(jnp.bfloat16),
      ep_w2, ep_b2)
    edge_partner_logits = partner.reshape(n)

    par_nf = node_features[partner_index_values]
    el_precat = jnp.concatenate([el_pre, el_w1_post], axis=0).astype(jnp.bfloat16)
    pidx2 = partner_index_index.astype(jnp.int32).reshape(p, 1)

    label = pl.pallas_call(
        functools.partial(_label_kernel, num_graphs=b),
        out_shape=jax.ShapeDtypeStruct((p, NUM_EDGE_TYPES), jnp.float32),
        grid=(pl.cdiv(p, P_TILE),),
        in_specs=[
            pl.BlockSpec((P_TILE, 1), lambda i: (i, 0)),
            pl.BlockSpec((P_TILE, f), lambda i: (i, 0)),
            _pinned((b + e, e)),
            _pinned((f, e)),
            _pinned((1, e)),
            _pinned((e, e)),
            _pinned((1, e)),
            _pinned((e, NUM_EDGE_TYPES)),
            _pinned((1, NUM_EDGE_TYPES)),
        ],
        out_specs=pl.BlockSpec((P_TILE, NUM_EDGE_TYPES), lambda i: (i, 0)),
        compiler_params=pltpu.CompilerParams(
            dimension_semantics=("parallel",),
            vmem_limit_bytes=64 * 1024 * 1024,
        ),
    )(pidx2, par_nf, el_precat, core_w.astype(jnp.bfloat16), core_b,
      el_w2.astype(jnp.bfloat16), el_b2,
      el_w3.astype(jnp.bfloat16), el_b3)

    return {"edge_partner_logits": edge_partner_logits,
            "edge_label_logits": label}


# speedup vs baseline: 6.0267x; 1.0009x over previous
import functools

import jax
import jax.numpy as jnp
from jax import lax
from jax.experimental import pallas as pl
from jax.experimental.pallas import tpu as pltpu

EMBED_DIM = 32
NODE_FEAT_DIM = 16
NUM_EDGE_TYPES = 8
ROW_TILE = 4096
SUB = 128
WIN = 256
P_TILE = 2048


def _pinned(shape):
    return pl.BlockSpec(shape, lambda i: tuple(0 for _ in shape))


def _onehot_w(gid_row, wstart, biota, s):
    lgid = (gid_row[:, s * SUB:(s + 1) * SUB] - wstart).astype(jnp.bfloat16)
    return (biota == lgid).astype(jnp.bfloat16)


def _segsum_kernel(widx_ref, gidr_ref, xt_ref, out_ref,
                   *, n_sub, steps_per_core):
    c = pl.program_id(0)
    i = pl.program_id(1)

    @pl.when(i == 0)
    def _():
        out_ref[...] = jnp.zeros_like(out_ref)

    xt = xt_ref[...]
    gid_row = gidr_ref[0]
    base = (c * steps_per_core + i) * n_sub
    biota = lax.broadcasted_iota(jnp.int32, (WIN, SUB), 0).astype(jnp.bfloat16)
    for s in range(n_sub):
        wstart = pl.multiple_of(widx_ref[base + s], SUB)
        oh_w = _onehot_w(gid_row, wstart, biota, s)
        part = lax.dot_general(xt[:, s * SUB:(s + 1) * SUB], oh_w,
                               (((1,), (1,)), ((), ())),
                               preferred_element_type=jnp.float32)
        out_ref[0, :, pl.ds(wstart, WIN)] += part


def _partner_kernel(widx_ref, gidr_ref, xt_ref, eppret_ref, w1pt_ref,
                    w2_ref, b2_ref, out_ref, *, n_sub):
    i = pl.program_id(0)
    xt = xt_ref[...]
    xw = jnp.dot(w1pt_ref[...], xt, preferred_element_type=jnp.float32)
    gid_row = gidr_ref[0]
    biota = lax.broadcasted_iota(jnp.int32, (WIN, SUB), 0).astype(jnp.bfloat16)
    pres = []
    for s in range(n_sub):
        wstart = pl.multiple_of(widx_ref[i * n_sub + s], SUB)
        oh_w = _onehot_w(gid_row, wstart, biota, s)
        pres.append(jnp.dot(eppret_ref[:, pl.ds(wstart, WIN)], oh_w,
                            preferred_element_type=jnp.float32))
    pre_t = jnp.concatenate(pres, axis=1)
    h_t = jnp.maximum(xw + pre_t, 0.0).astype(jnp.bfloat16)
    row = lax.dot_general(w2_ref[...], h_t, (((0,), (0,)), ((), ())),
                          preferred_element_type=jnp.float32)
    out_ref[0] = row + b2_ref[...]


def _label_kernel(pidx_ref, pnf_ref, precat_ref, corew_ref, coreb_ref,
                  w2_ref, b2_ref, w3_ref, b3_ref, out_ref, *, num_graphs):
    tile = pnf_ref.shape[0]
    par = jnp.maximum(
        jnp.dot(pnf_ref[...].astype(jnp.bfloat16), corew_ref[...],
                preferred_element_type=jnp.float32) + coreb_ref[...], 0.0)
    pidx = pidx_ref[...]
    giota = lax.broadcasted_iota(jnp.int32, (tile, num_graphs), 1)
    onehot = (pidx == giota).astype(jnp.bfloat16)
    lhs = jnp.concatenate([onehot, par.astype(jnp.bfloat16)], axis=1)
    h1 = jnp.maximum(
        jnp.dot(lhs, precat_ref[...], preferred_element_type=jnp.float32), 0.0)
    h2 = jnp.maximum(
        jnp.dot(h1.astype(jnp.bfloat16), w2_ref[...],
                preferred_element_type=jnp.float32) + b2_ref[...], 0.0)
    out_ref[...] = (jnp.dot(h2.astype(jnp.bfloat16), w3_ref[...],
                            preferred_element_type=jnp.float32) + b3_ref[...])


def kernel(core_w, core_b, core_wg, core_bg, ep_w1_post, ep_w1_cur,
           ep_w1_glob, ep_b1, el_w1_cur, el_w1_post, el_w1_glob, el_b1,
           el_w2, el_b2, el_w3, el_b3, wout_el, wout_ep, bout,
           node_counts, node_offsets, graph_ids, node_features,
           partner_index_index, partner_index_values):
    n, f = node_features.shape
    e = EMBED_DIM
    b = node_counts.shape[0]
    p = partner_index_index.shape[0]
    hp = lax.Precision.HIGHEST

    gids = graph_ids.astype(jnp.int32)

    xt0 = lax.dot_general(core_w, node_features, (((0,), (1,)), ((), ())))
    xt_bf = jnp.maximum(xt0 + core_b.T, 0.0).astype(jnp.bfloat16)

    w_idx = jnp.minimum((gids[::SUB] // SUB) * SUB, b - WIN)

    n_steps = n // ROW_TILE
    steps_per_core = n_steps // 2
    n_sub = ROW_TILE // SUB
    gid_rows = gids.reshape(n_steps, 1, ROW_TILE)

    seg_partial = pl.pallas_call(
        functools.partial(_segsum_kernel, n_sub=n_sub,
                          steps_per_core=steps_per_core),
        grid_spec=pltpu.PrefetchScalarGridSpec(
            num_scalar_prefetch=1,
            grid=(2, steps_per_core),
            in_specs=[
                pl.BlockSpec((1, 1, ROW_TILE),
                             lambda c, i, w: (c * steps_per_core + i, 0, 0)),
                pl.BlockSpec((e, ROW_TILE),
                             lambda c, i, w: (0, c * steps_per_core + i)),
            ],
            out_specs=pl.BlockSpec((1, e, b), lambda c, i, w: (c, 0, 0)),
        ),
        out_shape=jax.ShapeDtypeStruct((2, e, b), jnp.float32),
        compiler_params=pltpu.CompilerParams(
            dimension_semantics=("parallel", "arbitrary"),
            vmem_limit_bytes=64 * 1024 * 1024,
        ),
    )(w_idx, gid_rows, xt_bf)
    seg_sum = (seg_partial[0] + seg_partial[1]).T

    counts = jnp.maximum(node_counts.astype(jnp.float32), 1.0)[:, None]
    global_emb = (seg_sum / counts) @ core_wg + core_bg

    last_nf = node_features[node_offsets[1:] - 1]
    node_current = jnp.maximum(last_nf @ core_w + core_b, 0.0)
    ep_pre = (jnp.dot(node_current, ep_w1_cur, precision=hp)
              + jnp.dot(global_emb, ep_w1_glob, precision=hp) + ep_b1)
    el_pre = (jnp.dot(node_current, el_w1_cur, precision=hp)
              + jnp.dot(global_emb, el_w1_glob, precision=hp) + el_b1)

    ep_w2 = wout_ep[:, NUM_EDGE_TYPES:NUM_EDGE_TYPES + 1].astype(jnp.bfloat16)
    ep_b2 = bout[:, NUM_EDGE_TYPES:NUM_EDGE_TYPES + 1]

    partner = pl.pallas_call(
        functools.partial(_partner_kernel, n_sub=n_sub),
        grid_spec=pltpu.PrefetchScalarGridSpec(
            num_scalar_prefetch=1,
            grid=(n_steps,),
            in_specs=[
                pl.BlockSpec((1, 1, ROW_TILE), lambda i, w: (i, 0, 0)),
                pl.BlockSpec((e, ROW_TILE), lambda i, w: (0, i)),
                pl.BlockSpec((e, b), lambda i, w: (0, 0)),
                pl.BlockSpec((e, e), lambda i, w: (0, 0)),
                pl.BlockSpec((e, 1), lambda i, w: (0, 0)),
                pl.BlockSpec((1, 1), lambda i, w: (0, 0)),
            ],
            out_specs=pl.BlockSpec((1, 1, ROW_TILE), lambda i, w: (i, 0, 0)),
        ),
        out_shape=jax.ShapeDtypeStruct((n_steps, 1, ROW_TILE), jnp.float32),
        compiler_params=pltpu.CompilerParams(
            dimension_semantics=("parallel",),
            vmem_limit_bytes=64 * 1024 * 1024,
        ),
    )(w_idx, gid_rows, xt_bf,
      ep_pre.T.astype(jnp.bfloat16), ep_w1_post.T.astype(jnp.bfloat16),
      ep_w2, ep_b2)
    edge_partner_logits = partner.reshape(n)

    par_nf = node_features[partner_index_values]
    el_precat = jnp.concatenate([el_pre, el_w1_post], axis=0).astype(jnp.bfloat16)
    pidx2 = partner_index_index.astype(jnp.int32).reshape(p, 1)

    label = pl.pallas_call(
        functools.partial(_label_kernel, num_graphs=b),
        out_shape=jax.ShapeDtypeStruct((p, NUM_EDGE_TYPES), jnp.float32),
        grid=(pl.cdiv(p, P_TILE),),
        in_specs=[
            pl.BlockSpec((P_TILE, 1), lambda i: (i, 0)),
            pl.BlockSpec((P_TILE, f), lambda i: (i, 0)),
            _pinned((b + e, e)),
            _pinned((f, e)),
            _pinned((1, e)),
            _pinned((e, e)),
            _pinned((1, e)),
            _pinned((e, NUM_EDGE_TYPES)),
            _pinned((1, NUM_EDGE_TYPES)),
        ],
        out_specs=pl.BlockSpec((P_TILE, NUM_EDGE_TYPES), lambda i: (i, 0)),
        compiler_params=pltpu.CompilerParams(
            dimension_semantics=("parallel",),
            vmem_limit_bytes=64 * 1024 * 1024,
        ),
    )(pidx2, par_nf, el_precat, core_w.astype(jnp.bfloat16), core_b,
      el_w2.astype(jnp.bfloat16), el_b2,
      el_w3.astype(jnp.bfloat16), el_b3)

    return {"edge_partner_logits": edge_partner_logits,
            "edge_label_logits": label}


# speedup vs baseline: 6.4255x; 1.0662x over previous
import functools

import jax
import jax.numpy as jnp
from jax import lax
from jax.experimental import pallas as pl
from jax.experimental.pallas import tpu as pltpu

EMBED_DIM = 32
NODE_FEAT_DIM = 16
NUM_EDGE_TYPES = 8
ROW_TILE = 8192
SUB = 128
WIN = 256
P_TILE = 2048


def _pinned(shape):
    return pl.BlockSpec(shape, lambda i: tuple(0 for _ in shape))


def _onehot_w(gid_row, wstart, biota, s):
    lgid = (gid_row[:, s * SUB:(s + 1) * SUB] - wstart).astype(jnp.bfloat16)
    return (biota == lgid).astype(jnp.bfloat16)


def _segsum_kernel(widx_ref, gidr_ref, xt_ref, out_ref,
                   *, n_sub, steps_per_core):
    c = pl.program_id(0)
    i = pl.program_id(1)

    @pl.when(i == 0)
    def _():
        out_ref[...] = jnp.zeros_like(out_ref)

    xt = xt_ref[...]
    gid_row = gidr_ref[0]
    base = (c * steps_per_core + i) * n_sub
    biota = lax.broadcasted_iota(jnp.int32, (WIN, SUB), 0).astype(jnp.bfloat16)
    for s in range(n_sub):
        wstart = pl.multiple_of(widx_ref[base + s], SUB)
        oh_w = _onehot_w(gid_row, wstart, biota, s)
        part = lax.dot_general(xt[:, s * SUB:(s + 1) * SUB], oh_w,
                               (((1,), (1,)), ((), ())),
                               preferred_element_type=jnp.float32)
        out_ref[0, :, pl.ds(wstart, WIN)] += part


def _partner_kernel(widx_ref, gidr_ref, xt_ref, eppret_ref, w1pt_ref,
                    w2_ref, b2_ref, out_ref, *, n_sub):
    i = pl.program_id(0)
    xt = xt_ref[...]
    xw = jnp.dot(w1pt_ref[...], xt, preferred_element_type=jnp.float32)
    gid_row = gidr_ref[0]
    biota = lax.broadcasted_iota(jnp.int32, (WIN, SUB), 0).astype(jnp.bfloat16)
    pres = []
    for s in range(n_sub):
        wstart = pl.multiple_of(widx_ref[i * n_sub + s], SUB)
        oh_w = _onehot_w(gid_row, wstart, biota, s)
        pres.append(jnp.dot(eppret_ref[:, pl.ds(wstart, WIN)], oh_w,
                            preferred_element_type=jnp.float32))
    pre_t = jnp.concatenate(pres, axis=1)
    h_t = jnp.maximum(xw + pre_t, 0.0).astype(jnp.bfloat16)
    row = lax.dot_general(w2_ref[...], h_t, (((0,), (0,)), ((), ())),
                          preferred_element_type=jnp.float32)
    out_ref[0] = row + b2_ref[...]


def _label_kernel(pidx_ref, pnf_ref, precat_ref, corew_ref, coreb_ref,
                  w2_ref, b2_ref, w3_ref, b3_ref, out_ref, *, num_graphs):
    tile = pnf_ref.shape[0]
    par = jnp.maximum(
        jnp.dot(pnf_ref[...].astype(jnp.bfloat16), corew_ref[...],
                preferred_element_type=jnp.float32) + coreb_ref[...], 0.0)
    pidx = pidx_ref[...]
    giota = lax.broadcasted_iota(jnp.int32, (tile, num_graphs), 1)
    onehot = (pidx == giota).astype(jnp.bfloat16)
    lhs = jnp.concatenate([onehot, par.astype(jnp.bfloat16)], axis=1)
    h1 = jnp.maximum(
        jnp.dot(lhs, precat_ref[...], preferred_element_type=jnp.float32), 0.0)
    h2 = jnp.maximum(
        jnp.dot(h1.astype(jnp.bfloat16), w2_ref[...],
                preferred_element_type=jnp.float32) + b2_ref[...], 0.0)
    out_ref[...] = (jnp.dot(h2.astype(jnp.bfloat16), w3_ref[...],
                            preferred_element_type=jnp.float32) + b3_ref[...])


def kernel(core_w, core_b, core_wg, core_bg, ep_w1_post, ep_w1_cur,
           ep_w1_glob, ep_b1, el_w1_cur, el_w1_post, el_w1_glob, el_b1,
           el_w2, el_b2, el_w3, el_b3, wout_el, wout_ep, bout,
           node_counts, node_offsets, graph_ids, node_features,
           partner_index_index, partner_index_values):
    n, f = node_features.shape
    e = EMBED_DIM
    b = node_counts.shape[0]
    p = partner_index_index.shape[0]
    hp = lax.Precision.HIGHEST

    gids = graph_ids.astype(jnp.int32)

    xt0 = lax.dot_general(core_w, node_features, (((0,), (1,)), ((), ())))
    xt_bf = jnp.maximum(xt0 + core_b.T, 0.0).astype(jnp.bfloat16)

    w_idx = jnp.minimum((gids[::SUB] // SUB) * SUB, b - WIN)

    n_steps = n // ROW_TILE
    steps_per_core = n_steps // 2
    n_sub = ROW_TILE // SUB
    gid_rows = gids.reshape(n_steps, 1, ROW_TILE)

    seg_partial = pl.pallas_call(
        functools.partial(_segsum_kernel, n_sub=n_sub,
                          steps_per_core=steps_per_core),
        grid_spec=pltpu.PrefetchScalarGridSpec(
            num_scalar_prefetch=1,
            grid=(2, steps_per_core),
            in_specs=[
                pl.BlockSpec((1, 1, ROW_TILE),
                             lambda c, i, w: (c * steps_per_core + i, 0, 0)),
                pl.BlockSpec((e, ROW_TILE),
                             lambda c, i, w: (0, c * steps_per_core + i)),
            ],
            out_specs=pl.BlockSpec((1, e, b), lambda c, i, w: (c, 0, 0)),
        ),
        out_shape=jax.ShapeDtypeStruct((2, e, b), jnp.float32),
        compiler_params=pltpu.CompilerParams(
            dimension_semantics=("parallel", "arbitrary"),
            vmem_limit_bytes=64 * 1024 * 1024,
        ),
    )(w_idx, gid_rows, xt_bf)
    seg_sum = (seg_partial[0] + seg_partial[1]).T

    counts = jnp.maximum(node_counts.astype(jnp.float32), 1.0)[:, None]
    global_emb = (seg_sum / counts) @ core_wg + core_bg

    last_nf = node_features[node_offsets[1:] - 1]
    node_current = jnp.maximum(last_nf @ core_w + core_b, 0.0)
    ep_pre = (jnp.dot(node_current, ep_w1_cur, precision=hp)
              + jnp.dot(global_emb, ep_w1_glob, precision=hp) + ep_b1)
    el_pre = (jnp.dot(node_current, el_w1_cur, precision=hp)
              + jnp.dot(global_emb, el_w1_glob, precision=hp) + el_b1)

    ep_w2 = wout_ep[:, NUM_EDGE_TYPES:NUM_EDGE_TYPES + 1].astype(jnp.bfloat16)
    ep_b2 = bout[:, NUM_EDGE_TYPES:NUM_EDGE_TYPES + 1]

    partner = pl.pallas_call(
        functools.partial(_partner_kernel, n_sub=n_sub),
        grid_spec=pltpu.PrefetchScalarGridSpec(
            num_scalar_prefetch=1,
            grid=(n_steps,),
            in_specs=[
                pl.BlockSpec((1, 1, ROW_TILE), lambda i, w: (i, 0, 0)),
                pl.BlockSpec((e, ROW_TILE), lambda i, w: (0, i)),
                pl.BlockSpec((e, b), lambda i, w: (0, 0)),
                pl.BlockSpec((e, e), lambda i, w: (0, 0)),
                pl.BlockSpec((e, 1), lambda i, w: (0, 0)),
                pl.BlockSpec((1, 1), lambda i, w: (0, 0)),
            ],
            out_specs=pl.BlockSpec((1, 1, ROW_TILE), lambda i, w: (i, 0, 0)),
        ),
        out_shape=jax.ShapeDtypeStruct((n_steps, 1, ROW_TILE), jnp.float32),
        compiler_params=pltpu.CompilerParams(
            dimension_semantics=("parallel",),
            vmem_limit_bytes=64 * 1024 * 1024,
        ),
    )(w_idx, gid_rows, xt_bf,
      ep_pre.T.astype(jnp.bfloat16), ep_w1_post.T.astype(jnp.bfloat16),
      ep_w2, ep_b2)
    edge_partner_logits = partner.reshape(n)

    par_nf = node_features[partner_index_values]
    el_precat = jnp.concatenate([el_pre, el_w1_post], axis=0).astype(jnp.bfloat16)
    pidx2 = partner_index_index.astype(jnp.int32).reshape(p, 1)

    label = pl.pallas_call(
        functools.partial(_label_kernel, num_graphs=b),
        out_shape=jax.ShapeDtypeStruct((p, NUM_EDGE_TYPES), jnp.float32),
        grid=(pl.cdiv(p, P_TILE),),
        in_specs=[
            pl.BlockSpec((P_TILE, 1), lambda i: (i, 0)),
            pl.BlockSpec((P_TILE, f), lambda i: (i, 0)),
            _pinned((b + e, e)),
            _pinned((f, e)),
            _pinned((1, e)),
            _pinned((e, e)),
            _pinned((1, e)),
            _pinned((e, NUM_EDGE_TYPES)),
            _pinned((1, NUM_EDGE_TYPES)),
        ],
        out_specs=pl.BlockSpec((P_TILE, NUM_EDGE_TYPES), lambda i: (i, 0)),
        compiler_params=pltpu.CompilerParams(
            dimension_semantics=("parallel",),
            vmem_limit_bytes=64 * 1024 * 1024,
        ),
    )(pidx2, par_nf, el_precat, core_w.astype(jnp.bfloat16), core_b,
      el_w2.astype(jnp.bfloat16), el_b2,
      el_w3.astype(jnp.bfloat16), el_b3)

    return {"edge_partner_logits": edge_partner_logits,
            "edge_label_logits": label}


# speedup vs baseline: 6.6593x; 1.0364x over previous
import functools

import jax
import jax.numpy as jnp
from jax import lax
from jax.experimental import pallas as pl
from jax.experimental.pallas import tpu as pltpu

EMBED_DIM = 32
NODE_FEAT_DIM = 16
NUM_EDGE_TYPES = 8
ROW_TILE = 16384
SUB = 128
WIN = 256
P_TILE = 2048


def _pinned(shape):
    return pl.BlockSpec(shape, lambda i: tuple(0 for _ in shape))


def _onehot_w(gid_row, wstart, biota, s):
    lgid = (gid_row[:, s * SUB:(s + 1) * SUB] - wstart).astype(jnp.bfloat16)
    return (biota == lgid).astype(jnp.bfloat16)


def _segsum_kernel(widx_ref, gidr_ref, xt_ref, out_ref,
                   *, n_sub, steps_per_core):
    c = pl.program_id(0)
    i = pl.program_id(1)

    @pl.when(i == 0)
    def _():
        out_ref[...] = jnp.zeros_like(out_ref)

    xt = xt_ref[...]
    gid_row = gidr_ref[0]
    base = (c * steps_per_core + i) * n_sub
    biota = lax.broadcasted_iota(jnp.int32, (WIN, SUB), 0).astype(jnp.bfloat16)
    for s in range(n_sub):
        wstart = pl.multiple_of(widx_ref[base + s], SUB)
        oh_w = _onehot_w(gid_row, wstart, biota, s)
        part = lax.dot_general(xt[:, s * SUB:(s + 1) * SUB], oh_w,
                               (((1,), (1,)), ((), ())),
                               preferred_element_type=jnp.float32)
        out_ref[0, :, pl.ds(wstart, WIN)] += part


def _partner_kernel(widx_ref, gidr_ref, xt_ref, eppret_ref, w1pt_ref,
                    w2_ref, b2_ref, out_ref, *, n_sub):
    i = pl.program_id(0)
    xt = xt_ref[...]
    xw = jnp.dot(w1pt_ref[...], xt, preferred_element_type=jnp.float32)
    gid_row = gidr_ref[0]
    biota = lax.broadcasted_iota(jnp.int32, (WIN, SUB), 0).astype(jnp.bfloat16)
    pres = []
    for s in range(n_sub):
        wstart = pl.multiple_of(widx_ref[i * n_sub + s], SUB)
        oh_w = _onehot_w(gid_row, wstart, biota, s)
        pres.append(jnp.dot(eppret_ref[:, pl.ds(wstart, WIN)], oh_w,
                            preferred_element_type=jnp.float32))
    pre_t = jnp.concatenate(pres, axis=1)
    h_t = jnp.maximum(xw + pre_t, 0.0).astype(jnp.bfloat16)
    row = lax.dot_general(w2_ref[...], h_t, (((0,), (0,)), ((), ())),
                          preferred_element_type=jnp.float32)
    out_ref[0] = row + b2_ref[...]


def _label_kernel(pidx_ref, pnf_ref, precat_ref, corew_ref, coreb_ref,
                  w2_ref, b2_ref, w3_ref, b3_ref, out_ref, *, num_graphs):
    tile = pnf_ref.shape[0]
    par = jnp.maximum(
        jnp.dot(pnf_ref[...].astype(jnp.bfloat16), corew_ref[...],
                preferred_element_type=jnp.float32) + coreb_ref[...], 0.0)
    pidx = pidx_ref[...]
    giota = lax.broadcasted_iota(jnp.int32, (tile, num_graphs), 1)
    onehot = (pidx == giota).astype(jnp.bfloat16)
    lhs = jnp.concatenate([onehot, par.astype(jnp.bfloat16)], axis=1)
    h1 = jnp.maximum(
        jnp.dot(lhs, precat_ref[...], preferred_element_type=jnp.float32), 0.0)
    h2 = jnp.maximum(
        jnp.dot(h1.astype(jnp.bfloat16), w2_ref[...],
                preferred_element_type=jnp.float32) + b2_ref[...], 0.0)
    out_ref[...] = (jnp.dot(h2.astype(jnp.bfloat16), w3_ref[...],
                            preferred_element_type=jnp.float32) + b3_ref[...])


def kernel(core_w, core_b, core_wg, core_bg, ep_w1_post, ep_w1_cur,
           ep_w1_glob, ep_b1, el_w1_cur, el_w1_post, el_w1_glob, el_b1,
           el_w2, el_b2, el_w3, el_b3, wout_el, wout_ep, bout,
           node_counts, node_offsets, graph_ids, node_features,
           partner_index_index, partner_index_values):
    n, f = node_features.shape
    e = EMBED_DIM
    b = node_counts.shape[0]
    p = partner_index_index.shape[0]
    hp = lax.Precision.HIGHEST

    gids = graph_ids.astype(jnp.int32)

    xt0 = lax.dot_general(core_w, node_features, (((0,), (1,)), ((), ())))
    xt_bf = jnp.maximum(xt0 + core_b.T, 0.0).astype(jnp.bfloat16)

    w_idx = jnp.minimum((gids[::SUB] // SUB) * SUB, b - WIN)

    n_steps = n // ROW_TILE
    steps_per_core = n_steps // 2
    n_sub = ROW_TILE // SUB
    gid_rows = gids.reshape(n_steps, 1, ROW_TILE)

    seg_partial = pl.pallas_call(
        functools.partial(_segsum_kernel, n_sub=n_sub,
                          steps_per_core=steps_per_core),
        grid_spec=pltpu.PrefetchScalarGridSpec(
            num_scalar_prefetch=1,
            grid=(2, steps_per_core),
            in_specs=[
                pl.BlockSpec((1, 1, ROW_TILE),
                             lambda c, i, w: (c * steps_per_core + i, 0, 0)),
                pl.BlockSpec((e, ROW_TILE),
                             lambda c, i, w: (0, c * steps_per_core + i)),
            ],
            out_specs=pl.BlockSpec((1, e, b), lambda c, i, w: (c, 0, 0)),
        ),
        out_shape=jax.ShapeDtypeStruct((2, e, b), jnp.float32),
        compiler_params=pltpu.CompilerParams(
            dimension_semantics=("parallel", "arbitrary"),
            vmem_limit_bytes=64 * 1024 * 1024,
        ),
    )(w_idx, gid_rows, xt_bf)
    seg_sum = (seg_partial[0] + seg_partial[1]).T

    counts = jnp.maximum(node_counts.astype(jnp.float32), 1.0)[:, None]
    global_emb = (seg_sum / counts) @ core_wg + core_bg

    last_nf = node_features[node_offsets[1:] - 1]
    node_current = jnp.maximum(last_nf @ core_w + core_b, 0.0)
    ep_pre = (jnp.dot(node_current, ep_w1_cur, precision=hp)
              + jnp.dot(global_emb, ep_w1_glob, precision=hp) + ep_b1)
    el_pre = (jnp.dot(node_current, el_w1_cur, precision=hp)
              + jnp.dot(global_emb, el_w1_glob, precision=hp) + el_b1)

    ep_w2 = wout_ep[:, NUM_EDGE_TYPES:NUM_EDGE_TYPES + 1].astype(jnp.bfloat16)
    ep_b2 = bout[:, NUM_EDGE_TYPES:NUM_EDGE_TYPES + 1]

    partner = pl.pallas_call(
        functools.partial(_partner_kernel, n_sub=n_sub),
        grid_spec=pltpu.PrefetchScalarGridSpec(
            num_scalar_prefetch=1,
            grid=(n_steps,),
            in_specs=[
                pl.BlockSpec((1, 1, ROW_TILE), lambda i, w: (i, 0, 0)),
                pl.BlockSpec((e, ROW_TILE), lambda i, w: (0, i)),
                pl.BlockSpec((e, b), lambda i, w: (0, 0)),
                pl.BlockSpec((e, e), lambda i, w: (0, 0)),
                pl.BlockSpec((e, 1), lambda i, w: (0, 0)),
                pl.BlockSpec((1, 1), lambda i, w: (0, 0)),
            ],
            out_specs=pl.BlockSpec((1, 1, ROW_TILE), lambda i, w: (i, 0, 0)),
        ),
        out_shape=jax.ShapeDtypeStruct((n_steps, 1, ROW_TILE), jnp.float32),
        compiler_params=pltpu.CompilerParams(
            dimension_semantics=("parallel",),
            vmem_limit_bytes=64 * 1024 * 1024,
        ),
    )(w_idx, gid_rows, xt_bf,
      ep_pre.T.astype(jnp.bfloat16), ep_w1_post.T.astype(jnp.bfloat16),
      ep_w2, ep_b2)
    edge_partner_logits = partner.reshape(n)

    par_nf = node_features[partner_index_values]
    el_precat = jnp.concatenate([el_pre, el_w1_post], axis=0).astype(jnp.bfloat16)
    pidx2 = partner_index_index.astype(jnp.int32).reshape(p, 1)

    label = pl.pallas_call(
        functools.partial(_label_kernel, num_graphs=b),
        out_shape=jax.ShapeDtypeStruct((p, NUM_EDGE_TYPES), jnp.float32),
        grid=(pl.cdiv(p, P_TILE),),
        in_specs=[
            pl.BlockSpec((P_TILE, 1), lambda i: (i, 0)),
            pl.BlockSpec((P_TILE, f), lambda i: (i, 0)),
            _pinned((b + e, e)),
            _pinned((f, e)),
            _pinned((1, e)),
            _pinned((e, e)),
            _pinned((1, e)),
            _pinned((e, NUM_EDGE_TYPES)),
            _pinned((1, NUM_EDGE_TYPES)),
        ],
        out_specs=pl.BlockSpec((P_TILE, NUM_EDGE_TYPES), lambda i: (i, 0)),
        compiler_params=pltpu.CompilerParams(
            dimension_semantics=("parallel",),
            vmem_limit_bytes=64 * 1024 * 1024,
        ),
    )(pidx2, par_nf, el_precat, core_w.astype(jnp.bfloat16), core_b,
      el_w2.astype(jnp.bfloat16), el_b2,
      el_w3.astype(jnp.bfloat16), el_b3)

    return {"edge_partner_logits": edge_partner_logits,
            "edge_label_logits": label}


# speedup vs baseline: 6.6648x; 1.0008x over previous
import functools

import jax
import jax.numpy as jnp
from jax import lax
from jax.experimental import pallas as pl
from jax.experimental.pallas import tpu as pltpu

EMBED_DIM = 32
NODE_FEAT_DIM = 16
NUM_EDGE_TYPES = 8
ROW_TILE = 32768
SUB = 128
WIN = 256
P_TILE = 2048


def _pinned(shape):
    return pl.BlockSpec(shape, lambda i: tuple(0 for _ in shape))


def _onehot_w(gid_row, wstart, biota, s):
    lgid = (gid_row[:, s * SUB:(s + 1) * SUB] - wstart).astype(jnp.bfloat16)
    return (biota == lgid).astype(jnp.bfloat16)


def _segsum_kernel(widx_ref, gidr_ref, xt_ref, out_ref,
                   *, n_sub, steps_per_core):
    c = pl.program_id(0)
    i = pl.program_id(1)

    @pl.when(i == 0)
    def _():
        out_ref[...] = jnp.zeros_like(out_ref)

    xt = xt_ref[...]
    gid_row = gidr_ref[0]
    base = (c * steps_per_core + i) * n_sub
    biota = lax.broadcasted_iota(jnp.int32, (WIN, SUB), 0).astype(jnp.bfloat16)
    for s in range(n_sub):
        wstart = pl.multiple_of(widx_ref[base + s], SUB)
        oh_w = _onehot_w(gid_row, wstart, biota, s)
        part = lax.dot_general(xt[:, s * SUB:(s + 1) * SUB], oh_w,
                               (((1,), (1,)), ((), ())),
                               preferred_element_type=jnp.float32)
        out_ref[0, :, pl.ds(wstart, WIN)] += part


def _partner_kernel(widx_ref, gidr_ref, xt_ref, eppret_ref, w1pt_ref,
                    w2_ref, b2_ref, out_ref, *, n_sub):
    i = pl.program_id(0)
    xt = xt_ref[...]
    xw = jnp.dot(w1pt_ref[...], xt, preferred_element_type=jnp.float32)
    gid_row = gidr_ref[0]
    biota = lax.broadcasted_iota(jnp.int32, (WIN, SUB), 0).astype(jnp.bfloat16)
    pres = []
    for s in range(n_sub):
        wstart = pl.multiple_of(widx_ref[i * n_sub + s], SUB)
        oh_w = _onehot_w(gid_row, wstart, biota, s)
        pres.append(jnp.dot(eppret_ref[:, pl.ds(wstart, WIN)], oh_w,
                            preferred_element_type=jnp.float32))
    pre_t = jnp.concatenate(pres, axis=1)
    h_t = jnp.maximum(xw + pre_t, 0.0).astype(jnp.bfloat16)
    row = lax.dot_general(w2_ref[...], h_t, (((0,), (0,)), ((), ())),
                          preferred_element_type=jnp.float32)
    out_ref[0] = row + b2_ref[...]


def _label_kernel(pidx_ref, pnf_ref, precat_ref, corew_ref, coreb_ref,
                  w2_ref, b2_ref, w3_ref, b3_ref, out_ref, *, num_graphs):
    tile = pnf_ref.shape[0]
    par = jnp.maximum(
        jnp.dot(pnf_ref[...].astype(jnp.bfloat16), corew_ref[...],
                preferred_element_type=jnp.float32) + coreb_ref[...], 0.0)
    pidx = pidx_ref[...]
    giota = lax.broadcasted_iota(jnp.int32, (tile, num_graphs), 1)
    onehot = (pidx == giota).astype(jnp.bfloat16)
    lhs = jnp.concatenate([onehot, par.astype(jnp.bfloat16)], axis=1)
    h1 = jnp.maximum(
        jnp.dot(lhs, precat_ref[...], preferred_element_type=jnp.float32), 0.0)
    h2 = jnp.maximum(
        jnp.dot(h1.astype(jnp.bfloat16), w2_ref[...],
                preferred_element_type=jnp.float32) + b2_ref[...], 0.0)
    out_ref[...] = (jnp.dot(h2.astype(jnp.bfloat16), w3_ref[...],
                            preferred_element_type=jnp.float32) + b3_ref[...])


def kernel(core_w, core_b, core_wg, core_bg, ep_w1_post, ep_w1_cur,
           ep_w1_glob, ep_b1, el_w1_cur, el_w1_post, el_w1_glob, el_b1,
           el_w2, el_b2, el_w3, el_b3, wout_el, wout_ep, bout,
           node_counts, node_offsets, graph_ids, node_features,
           partner_index_index, partner_index_values):
    n, f = node_features.shape
    e = EMBED_DIM
    b = node_counts.shape[0]
    p = partner_index_index.shape[0]
    hp = lax.Precision.HIGHEST

    gids = graph_ids.astype(jnp.int32)

    xt0 = lax.dot_general(core_w, node_features, (((0,), (1,)), ((), ())))
    xt_bf = jnp.maximum(xt0 + core_b.T, 0.0).astype(jnp.bfloat16)

    w_idx = jnp.minimum((gids[::SUB] // SUB) * SUB, b - WIN)

    n_steps = n // ROW_TILE
    steps_per_core = n_steps // 2
    n_sub = ROW_TILE // SUB
    gid_rows = gids.reshape(n_steps, 1, ROW_TILE)

    seg_partial = pl.pallas_call(
        functools.partial(_segsum_kernel, n_sub=n_sub,
                          steps_per_core=steps_per_core),
        grid_spec=pltpu.PrefetchScalarGridSpec(
            num_scalar_prefetch=1,
            grid=(2, steps_per_core),
            in_specs=[
                pl.BlockSpec((1, 1, ROW_TILE),
                             lambda c, i, w: (c * steps_per_core + i, 0, 0)),
                pl.BlockSpec((e, ROW_TILE),
                             lambda c, i, w: (0, c * steps_per_core + i)),
            ],
            out_specs=pl.BlockSpec((1, e, b), lambda c, i, w: (c, 0, 0)),
        ),
        out_shape=jax.ShapeDtypeStruct((2, e, b), jnp.float32),
        compiler_params=pltpu.CompilerParams(
            dimension_semantics=("parallel", "arbitrary"),
            vmem_limit_bytes=64 * 1024 * 1024,
        ),
    )(w_idx, gid_rows, xt_bf)
    seg_sum = (seg_partial[0] + seg_partial[1]).T

    counts = jnp.maximum(node_counts.astype(jnp.float32), 1.0)[:, None]
    global_emb = (seg_sum / counts) @ core_wg + core_bg

    last_nf = node_features[node_offsets[1:] - 1]
    node_current = jnp.maximum(last_nf @ core_w + core_b, 0.0)
    ep_pre = (jnp.dot(node_current, ep_w1_cur, precision=hp)
              + jnp.dot(global_emb, ep_w1_glob, precision=hp) + ep_b1)
    el_pre = (jnp.dot(node_current, el_w1_cur, precision=hp)
              + jnp.dot(global_emb, el_w1_glob, precision=hp) + el_b1)

    ep_w2 = wout_ep[:, NUM_EDGE_TYPES:NUM_EDGE_TYPES + 1].astype(jnp.bfloat16)
    ep_b2 = bout[:, NUM_EDGE_TYPES:NUM_EDGE_TYPES + 1]

    partner = pl.pallas_call(
        functools.partial(_partner_kernel, n_sub=n_sub),
        grid_spec=pltpu.PrefetchScalarGridSpec(
            num_scalar_prefetch=1,
            grid=(n_steps,),
            in_specs=[
                pl.BlockSpec((1, 1, ROW_TILE), lambda i, w: (i, 0, 0)),
                pl.BlockSpec((e, ROW_TILE), lambda i, w: (0, i)),
                pl.BlockSpec((e, b), lambda i, w: (0, 0)),
                pl.BlockSpec((e, e), lambda i, w: (0, 0)),
                pl.BlockSpec((e, 1), lambda i, w: (0, 0)),
                pl.BlockSpec((1, 1), lambda i, w: (0, 0)),
            ],
            out_specs=pl.BlockSpec((1, 1, ROW_TILE), lambda i, w: (i, 0, 0)),
        ),
        out_shape=jax.ShapeDtypeStruct((n_steps, 1, ROW_TILE), jnp.float32),
        compiler_params=pltpu.CompilerParams(
            dimension_semantics=("parallel",),
            vmem_limit_bytes=64 * 1024 * 1024,
        ),
    )(w_idx, gid_rows, xt_bf,
      ep_pre.T.astype(jnp.bfloat16), ep_w1_post.T.astype(jnp.bfloat16),
      ep_w2, ep_b2)
    edge_partner_logits = partner.reshape(n)

    par_nf = node_features[partner_index_values]
    el_precat = jnp.concatenate([el_pre, el_w1_post], axis=0).astype(jnp.bfloat16)
    pidx2 = partner_index_index.astype(jnp.int32).reshape(p, 1)

    label = pl.pallas_call(
        functools.partial(_label_kernel, num_graphs=b),
        out_shape=jax.ShapeDtypeStruct((p, NUM_EDGE_TYPES), jnp.float32),
        grid=(pl.cdiv(p, P_TILE),),
        in_specs=[
            pl.BlockSpec((P_TILE, 1), lambda i: (i, 0)),
            pl.BlockSpec((P_TILE, f), lambda i: (i, 0)),
            _pinned((b + e, e)),
            _pinned((f, e)),
            _pinned((1, e)),
            _pinned((e, e)),
            _pinned((1, e)),
            _pinned((e, NUM_EDGE_TYPES)),
            _pinned((1, NUM_EDGE_TYPES)),
        ],
        out_specs=pl.BlockSpec((P_TILE, NUM_EDGE_TYPES), lambda i: (i, 0)),
        compiler_params=pltpu.CompilerParams(
            dimension_semantics=("parallel",),
            vmem_limit_bytes=64 * 1024 * 1024,
        ),
    )(pidx2, par_nf, el_precat, core_w.astype(jnp.bfloat16), core_b,
      el_w2.astype(jnp.bfloat16), el_b2,
      el_w3.astype(jnp.bfloat16), el_b3)

    return {"edge_partner_logits": edge_partner_logits,
            "edge_label_logits": label}


# speedup vs baseline: 6.7174x; 1.0079x over previous
import functools

import jax
import jax.numpy as jnp
from jax import lax
from jax.experimental import pallas as pl
from jax.experimental.pallas import tpu as pltpu

EMBED_DIM = 32
NODE_FEAT_DIM = 16
NUM_EDGE_TYPES = 8
ROW_TILE = 32768
SUB = 128
WIN = 256
P_TILE = 8192


def _pinned(shape):
    return pl.BlockSpec(shape, lambda i: tuple(0 for _ in shape))


def _onehot_w(gid_row, wstart, biota, s):
    lgid = (gid_row[:, s * SUB:(s + 1) * SUB] - wstart).astype(jnp.bfloat16)
    return (biota == lgid).astype(jnp.bfloat16)


def _segsum_kernel(widx_ref, gidr_ref, xt_ref, out_ref,
                   *, n_sub, steps_per_core):
    c = pl.program_id(0)
    i = pl.program_id(1)

    @pl.when(i == 0)
    def _():
        out_ref[...] = jnp.zeros_like(out_ref)

    xt = xt_ref[...]
    gid_row = gidr_ref[0]
    base = (c * steps_per_core + i) * n_sub
    biota = lax.broadcasted_iota(jnp.int32, (WIN, SUB), 0).astype(jnp.bfloat16)
    for s in range(n_sub):
        wstart = pl.multiple_of(widx_ref[base + s], SUB)
        oh_w = _onehot_w(gid_row, wstart, biota, s)
        part = lax.dot_general(xt[:, s * SUB:(s + 1) * SUB], oh_w,
                               (((1,), (1,)), ((), ())),
                               preferred_element_type=jnp.float32)
        out_ref[0, :, pl.ds(wstart, WIN)] += part


def _partner_kernel(widx_ref, gidr_ref, xt_ref, eppret_ref, w1pt_ref,
                    w2_ref, b2_ref, out_ref, *, n_sub):
    i = pl.program_id(0)
    xt = xt_ref[...]
    xw = jnp.dot(w1pt_ref[...], xt, preferred_element_type=jnp.float32)
    gid_row = gidr_ref[0]
    biota = lax.broadcasted_iota(jnp.int32, (WIN, SUB), 0).astype(jnp.bfloat16)
    pres = []
    for s in range(n_sub):
        wstart = pl.multiple_of(widx_ref[i * n_sub + s], SUB)
        oh_w = _onehot_w(gid_row, wstart, biota, s)
        pres.append(jnp.dot(eppret_ref[:, pl.ds(wstart, WIN)], oh_w,
                            preferred_element_type=jnp.float32))
    pre_t = jnp.concatenate(pres, axis=1)
    h_t = jnp.maximum(xw + pre_t, 0.0).astype(jnp.bfloat16)
    row = lax.dot_general(w2_ref[...], h_t, (((0,), (0,)), ((), ())),
                          preferred_element_type=jnp.float32)
    out_ref[0] = row + b2_ref[...]


def _label_kernel(pidx_ref, pnf_ref, precat_ref, corew_ref, coreb_ref,
                  w2_ref, b2_ref, w3_ref, b3_ref, out_ref, *, num_graphs):
    tile = pnf_ref.shape[0]
    par = jnp.maximum(
        jnp.dot(pnf_ref[...].astype(jnp.bfloat16), corew_ref[...],
                preferred_element_type=jnp.float32) + coreb_ref[...], 0.0)
    pidx = pidx_ref[...]
    giota = lax.broadcasted_iota(jnp.int32, (tile, num_graphs), 1)
    onehot = (pidx == giota).astype(jnp.bfloat16)
    lhs = jnp.concatenate([onehot, par.astype(jnp.bfloat16)], axis=1)
    h1 = jnp.maximum(
        jnp.dot(lhs, precat_ref[...], preferred_element_type=jnp.float32), 0.0)
    h2 = jnp.maximum(
        jnp.dot(h1.astype(jnp.bfloat16), w2_ref[...],
                preferred_element_type=jnp.float32) + b2_ref[...], 0.0)
    out_ref[...] = (jnp.dot(h2.astype(jnp.bfloat16), w3_ref[...],
                            preferred_element_type=jnp.float32) + b3_ref[...])


def kernel(core_w, core_b, core_wg, core_bg, ep_w1_post, ep_w1_cur,
           ep_w1_glob, ep_b1, el_w1_cur, el_w1_post, el_w1_glob, el_b1,
           el_w2, el_b2, el_w3, el_b3, wout_el, wout_ep, bout,
           node_counts, node_offsets, graph_ids, node_features,
           partner_index_index, partner_index_values):
    n, f = node_features.shape
    e = EMBED_DIM
    b = node_counts.shape[0]
    p = partner_index_index.shape[0]
    hp = lax.Precision.HIGHEST

    gids = graph_ids.astype(jnp.int32)

    xt0 = lax.dot_general(core_w, node_features, (((0,), (1,)), ((), ())))
    xt_bf = jnp.maximum(xt0 + core_b.T, 0.0).astype(jnp.bfloat16)

    w_idx = jnp.minimum((gids[::SUB] // SUB) * SUB, b - WIN)

    n_steps = n // ROW_TILE
    steps_per_core = n_steps // 2
    n_sub = ROW_TILE // SUB
    gid_rows = gids.reshape(n_steps, 1, ROW_TILE)

    seg_partial = pl.pallas_call(
        functools.partial(_segsum_kernel, n_sub=n_sub,
                          steps_per_core=steps_per_core),
        grid_spec=pltpu.PrefetchScalarGridSpec(
            num_scalar_prefetch=1,
            grid=(2, steps_per_core),
            in_specs=[
                pl.BlockSpec((1, 1, ROW_TILE),
                             lambda c, i, w: (c * steps_per_core + i, 0, 0)),
                pl.BlockSpec((e, ROW_TILE),
                             lambda c, i, w: (0, c * steps_per_core + i)),
            ],
            out_specs=pl.BlockSpec((1, e, b), lambda c, i, w: (c, 0, 0)),
        ),
        out_shape=jax.ShapeDtypeStruct((2, e, b), jnp.float32),
        compiler_params=pltpu.CompilerParams(
            dimension_semantics=("parallel", "arbitrary"),
            vmem_limit_bytes=64 * 1024 * 1024,
        ),
    )(w_idx, gid_rows, xt_bf)
    seg_sum = (seg_partial[0] + seg_partial[1]).T

    counts = jnp.maximum(node_counts.astype(jnp.float32), 1.0)[:, None]
    global_emb = (seg_sum / counts) @ core_wg + core_bg

    last_nf = node_features[node_offsets[1:] - 1]
    node_current = jnp.maximum(last_nf @ core_w + core_b, 0.0)
    w_cur = jnp.concatenate([ep_w1_cur, el_w1_cur], axis=1)
    w_glob = jnp.concatenate([ep_w1_glob, el_w1_glob], axis=1)
    b_pre = jnp.concatenate([ep_b1, el_b1], axis=1)
    pre_cat = (jnp.dot(node_current, w_cur, precision=hp)
               + jnp.dot(global_emb, w_glob, precision=hp) + b_pre)
    ep_pre = pre_cat[:, :e]
    el_pre = pre_cat[:, e:]

    ep_w2 = wout_ep[:, NUM_EDGE_TYPES:NUM_EDGE_TYPES + 1].astype(jnp.bfloat16)
    ep_b2 = bout[:, NUM_EDGE_TYPES:NUM_EDGE_TYPES + 1]

    partner = pl.pallas_call(
        functools.partial(_partner_kernel, n_sub=n_sub),
        grid_spec=pltpu.PrefetchScalarGridSpec(
            num_scalar_prefetch=1,
            grid=(n_steps,),
            in_specs=[
                pl.BlockSpec((1, 1, ROW_TILE), lambda i, w: (i, 0, 0)),
                pl.BlockSpec((e, ROW_TILE), lambda i, w: (0, i)),
                pl.BlockSpec((e, b), lambda i, w: (0, 0)),
                pl.BlockSpec((e, e), lambda i, w: (0, 0)),
                pl.BlockSpec((e, 1), lambda i, w: (0, 0)),
                pl.BlockSpec((1, 1), lambda i, w: (0, 0)),
            ],
            out_specs=pl.BlockSpec((1, 1, ROW_TILE), lambda i, w: (i, 0, 0)),
        ),
        out_shape=jax.ShapeDtypeStruct((n_steps, 1, ROW_TILE), jnp.float32),
        compiler_params=pltpu.CompilerParams(
            dimension_semantics=("parallel",),
            vmem_limit_bytes=64 * 1024 * 1024,
        ),
    )(w_idx, gid_rows, xt_bf,
      ep_pre.T.astype(jnp.bfloat16), ep_w1_post.T.astype(jnp.bfloat16),
      ep_w2, ep_b2)
    edge_partner_logits = partner.reshape(n)

    par_nf = node_features[partner_index_values]
    el_precat = jnp.concatenate([el_pre, el_w1_post], axis=0).astype(jnp.bfloat16)
    pidx2 = partner_index_index.astype(jnp.int32).reshape(p, 1)

    label = pl.pallas_call(
        functools.partial(_label_kernel, num_graphs=b),
        out_shape=jax.ShapeDtypeStruct((p, NUM_EDGE_TYPES), jnp.float32),
        grid=(pl.cdiv(p, P_TILE),),
        in_specs=[
            pl.BlockSpec((P_TILE, 1), lambda i: (i, 0)),
            pl.BlockSpec((P_TILE, f), lambda i: (i, 0)),
            _pinned((b + e, e)),
            _pinned((f, e)),
            _pinned((1, e)),
            _pinned((e, e)),
            _pinned((1, e)),
            _pinned((e, NUM_EDGE_TYPES)),
            _pinned((1, NUM_EDGE_TYPES)),
        ],
        out_specs=pl.BlockSpec((P_TILE, NUM_EDGE_TYPES), lambda i: (i, 0)),
        compiler_params=pltpu.CompilerParams(
            dimension_semantics=("parallel",),
            vmem_limit_bytes=64 * 1024 * 1024,
        ),
    )(pidx2, par_nf, el_precat, core_w.astype(jnp.bfloat16), core_b,
      el_w2.astype(jnp.bfloat16), el_b2,
      el_w3.astype(jnp.bfloat16), el_b3)

    return {"edge_partner_logits": edge_partner_logits,
            "edge_label_logits": label}
